# baseline (speedup 1.0000x reference)
import jax
import jax.numpy as jnp
from jax import lax
from jax.experimental import pallas as pl
from jax.experimental.pallas import tpu as pltpu
from jax.experimental.pallas import tpu_sc as plsc

N = 50000
S = 16
E = N * S // 2
D = 128
ED = 16
K = 8
NK = N * K
DW = D // 2
EDW = ED // 2

BN1 = 2000
NB1 = N // BN1


def _idx_f0_body(n2e_ref, sel_ref, feats_ref, wp_ref, idx_ref, f0_ref):
    sel = sel_ref[...]
    n2e = n2e_ref[...]
    acc = jnp.zeros(sel.shape, jnp.int32)
    for s in range(S):
        acc = jnp.where(sel == s, n2e[:, s:s + 1], acc)
    idx_ref[...] = acc
    f0_ref[...] = jnp.dot(feats_ref[...], wp_ref[...],
                          preferred_element_type=jnp.float32)


def _idx_f0(n2e, sel, feats, wp):
    return pl.pallas_call(
        _idx_f0_body,
        grid=(NB1,),
        in_specs=[
            pl.BlockSpec((BN1, S), lambda i: (i, 0)),
            pl.BlockSpec((BN1, K), lambda i: (i, 0)),
            pl.BlockSpec((BN1, D), lambda i: (i, 0)),
            pl.BlockSpec((D, D), lambda i: (0, 0)),
        ],
        out_specs=[
            pl.BlockSpec((BN1, K), lambda i: (i, 0)),
            pl.BlockSpec((BN1, D), lambda i: (i, 0)),
        ],
        out_shape=[
            jax.ShapeDtypeStruct((N, K), jnp.int32),
            jax.ShapeDtypeStruct((N, D), jnp.float32),
        ],
    )(n2e, sel, feats, wp)


def _wc_body(wp_ref, we0b_ref, wc_ref):
    wc_ref[...] = 0.5 * jnp.dot(wp_ref[...], we0b_ref[...],
                                preferred_element_type=jnp.float32)


def _wc(wp, we0b):
    return pl.pallas_call(
        _wc_body,
        out_shape=jax.ShapeDtypeStruct((D, D), jnp.float32),
    )(wp, we0b)


CH = 640
NCH = NK // CH
QR = CH // 128
_NC = 2
_NS = 16
_NW = _NC * _NS


def _gather_body(idx1_hbm, adj_hbm, featsw_hbm, embw_hbm,
                 ga_hbm, gb_hbm, gemb_hbm,
                 idx_v, av_v, bv_v, an_v, bn_v, rows_v, emb_v, sem):
    wid = lax.axis_index("s") * _NC + lax.axis_index("c")
    nt = (NCH - wid + _NW - 1) // _NW

    def body(t, carry):
        c = wid + t * _NW
        base = c * CH
        pltpu.sync_copy(idx1_hbm.at[pl.ds(base, CH)], idx_v)
        for j in range(CH // 16):
            v = idx_v[pl.ds(j * 16, 16)]
            av_v[pl.ds(j * 16, 16)] = v + v
            bv_v[pl.ds(j * 16, 16)] = v + v + 1
        cps = [pltpu.async_copy(adj_hbm.at[av_v.at[pl.ds(q * 128, 128)]],
                                an_v.at[pl.ds(q * 128, 128)], sem)
               for q in range(QR)]
        cps += [pltpu.async_copy(adj_hbm.at[bv_v.at[pl.ds(q * 128, 128)]],
                                 bn_v.at[pl.ds(q * 128, 128)], sem)
                for q in range(QR)]
        cps += [pltpu.async_copy(embw_hbm.at[idx_v.at[pl.ds(q * 128, 128)]],
                                 emb_v.at[pl.ds(q * 128, 128)], sem)
                for q in range(QR)]
        for cp in cps:
            cp.wait()
        pltpu.sync_copy(emb_v, gemb_hbm.at[pl.ds(base, CH)])
        cps = [pltpu.async_copy(featsw_hbm.at[an_v.at[pl.ds(q * 128, 128)]],
                                rows_v.at[pl.ds(q * 128, 128)], sem)
               for q in range(QR)]
        for cp in cps:
            cp.wait()
        pltpu.sync_copy(rows_v, ga_hbm.at[pl.ds(base, CH)])
        cps = [pltpu.async_copy(featsw_hbm.at[bn_v.at[pl.ds(q * 128, 128)]],
                                rows_v.at[pl.ds(q * 128, 128)], sem)
               for q in range(QR)]
        for cp in cps:
            cp.wait()
        pltpu.sync_copy(rows_v, gb_hbm.at[pl.ds(base, CH)])
        return carry

    lax.fori_loop(0, nt, body, 0)


def _gather_sc(idx1, adjf, featsw, embw):
    mesh = plsc.VectorSubcoreMesh(core_axis_name="c", subcore_axis_name="s")
    return pl.kernel(
        _gather_body,
        mesh=mesh,
        compiler_params=pltpu.CompilerParams(use_tc_tiling_on_sc=False),
        out_type=(
            jax.ShapeDtypeStruct((NK, DW), jnp.int32),
            jax.ShapeDtypeStruct((NK, DW), jnp.int32),
            jax.ShapeDtypeStruct((NK, EDW), jnp.int32),
        ),
        scratch_types=[
            pltpu.VMEM((CH,), jnp.int32),
            pltpu.VMEM((CH,), jnp.int32),
            pltpu.VMEM((CH,), jnp.int32),
            pltpu.VMEM((CH,), jnp.int32),
            pltpu.VMEM((CH,), jnp.int32),
            pltpu.VMEM((CH, DW), jnp.int32),
            pltpu.VMEM((CH, EDW), jnp.int32),
            pltpu.SemaphoreType.DMA,
        ],
    )(idx1, adjf, featsw, embw)


BN3 = 400
BS3 = BN3 * K
NB3 = N // BN3


def _heads_body(ga_ref, gb_ref, gemb_ref, f0_ref, wep_ref, we0a_ref, wc_ref,
                wn0a_ref, wn0b_ref, wn1a_ref, wn1b_ref, out_ref):
    ssum = (ga_ref[...].astype(jnp.float32)
            + gb_ref[...].astype(jnp.float32))
    e0 = jnp.dot(gemb_ref[...].astype(jnp.float32), wep_ref[...],
                 preferred_element_type=jnp.float32)
    eo = jnp.maximum(
        jnp.dot(e0, we0a_ref[...], preferred_element_type=jnp.float32)
        + jnp.dot(ssum, wc_ref[...], preferred_element_type=jnp.float32),
        0.0)
    em0 = jnp.sum(e0.reshape(BN3, K, D), axis=1) * (1.0 / K)
    em1 = jnp.sum(eo.reshape(BN3, K, D), axis=1) * (1.0 / K)
    f0 = f0_ref[...]
    f1 = jnp.maximum(
        jnp.dot(f0, wn0a_ref[...], preferred_element_type=jnp.float32)
        + jnp.dot(em0, wn0b_ref[...], preferred_element_type=jnp.float32),
        0.0)
    f2 = jnp.maximum(
        jnp.dot(f1, wn1a_ref[...], preferred_element_type=jnp.float32)
        + jnp.dot(em1, wn1b_ref[...], preferred_element_type=jnp.float32),
        0.0)
    out_ref[...] = jnp.concatenate([f1, f2], axis=-1)[None]


def _heads(ga, gb, gemb, f0, wep, we0a, wc, wn0a, wn0b, wn1a, wn1b):
    return pl.pallas_call(
        _heads_body,
        grid=(NB3,),
        in_specs=[
            pl.BlockSpec((BS3, D), lambda i: (i, 0)),
            pl.BlockSpec((BS3, D), lambda i: (i, 0)),
            pl.BlockSpec((BS3, ED), lambda i: (i, 0)),
            pl.BlockSpec((BN3, D), lambda i: (i, 0)),
            pl.BlockSpec((ED, D), lambda i: (0, 0)),
            pl.BlockSpec((D, D), lambda i: (0, 0)),
            pl.BlockSpec((D, D), lambda i: (0, 0)),
            pl.BlockSpec((D, D), lambda i: (0, 0)),
            pl.BlockSpec((D, D), lambda i: (0, 0)),
            pl.BlockSpec((D, D), lambda i: (0, 0)),
            pl.BlockSpec((D, D), lambda i: (0, 0)),
        ],
        out_specs=pl.BlockSpec((1, BN3, 2 * D), lambda i: (0, i, 0)),
        out_shape=jax.ShapeDtypeStruct((1, N, 2 * D), jnp.float32),
    )(ga, gb, gemb, f0, wep, we0a, wc, wn0a, wn0b, wn1a, wn1b)


def _pack_bf16(x, w):
    return lax.bitcast_convert_type(
        x.astype(jnp.bfloat16).reshape(x.shape[0], w, 2), jnp.int32)


def _unpack_bf16(xw, d):
    return lax.bitcast_convert_type(xw, jnp.bfloat16).reshape(xw.shape[0], d)


def kernel(feats, node2edge_idx, edge_emb, edge_node_adj, sel, W_prep,
           W_edge_prep, W_e0, W_n0, W_e1, W_n1):
    del W_e1
    idx, f0 = _idx_f0(node2edge_idx, sel, feats, W_prep)
    wc = _wc(W_prep, W_e0[D:])
    idx1 = idx.reshape(NK)
    adjf = edge_node_adj.reshape(2 * E)
    featsw = _pack_bf16(feats, DW)
    embw = _pack_bf16(edge_emb, EDW)
    gaw, gbw, gembw = _gather_sc(idx1, adjf, featsw, embw)
    ga = _unpack_bf16(gaw, D)
    gb = _unpack_bf16(gbw, D)
    gemb = _unpack_bf16(gembw, ED)
    return _heads(ga, gb, gemb, f0, W_edge_prep, W_e0[:D], wc,
                  W_n0[:D], W_n0[D:], W_n1[:D], W_n1[D:])


# speedup vs baseline: 2.1441x; 2.1441x over previous
import jax
import jax.numpy as jnp
from jax import lax
from jax.experimental import pallas as pl
from jax.experimental.pallas import tpu as pltpu
from jax.experimental.pallas import tpu_sc as plsc

N = 50000
S = 16
E = N * S // 2
D = 128
ED = 16
K = 8
NK = N * K
DW = D // 2
EDW = ED // 2

BN1 = 2000
NB1 = N // BN1


def _pack_halves(x, w):
    lo = lax.bitcast_convert_type(x[:, :w].astype(jnp.bfloat16), jnp.uint16)
    hi = lax.bitcast_convert_type(x[:, w:].astype(jnp.bfloat16), jnp.uint16)
    return (lo.astype(jnp.int32)
            | jnp.left_shift(hi.astype(jnp.int32), 16))


def _unpack_halves(xw):
    lo = lax.bitcast_convert_type(jnp.left_shift(xw, 16), jnp.float32)
    hi = lax.bitcast_convert_type(xw & jnp.int32(-65536), jnp.float32)
    return jnp.concatenate([lo, hi], axis=-1)


def _idx_f0_body(n2e_ref, sel_ref, feats_ref, wp_ref, idx_ref, f0_ref,
                 fw_ref):
    sel = sel_ref[...]
    n2e = n2e_ref[...]
    acc = jnp.zeros(sel.shape, jnp.int32)
    for s in range(S):
        acc = jnp.where(sel == s, n2e[:, s:s + 1], acc)
    idx_ref[...] = acc
    f = feats_ref[...]
    f0_ref[...] = jnp.dot(f, wp_ref[...], preferred_element_type=jnp.float32)
    fw_ref[...] = _pack_halves(f, DW)


def _idx_f0(n2e, sel, feats, wp):
    return pl.pallas_call(
        _idx_f0_body,
        grid=(NB1,),
        in_specs=[
            pl.BlockSpec((BN1, S), lambda i: (i, 0)),
            pl.BlockSpec((BN1, K), lambda i: (i, 0)),
            pl.BlockSpec((BN1, D), lambda i: (i, 0)),
            pl.BlockSpec((D, D), lambda i: (0, 0)),
        ],
        out_specs=[
            pl.BlockSpec((BN1, K), lambda i: (i, 0)),
            pl.BlockSpec((BN1, D), lambda i: (i, 0)),
            pl.BlockSpec((BN1, DW), lambda i: (i, 0)),
        ],
        out_shape=[
            jax.ShapeDtypeStruct((N, K), jnp.int32),
            jax.ShapeDtypeStruct((N, D), jnp.float32),
            jax.ShapeDtypeStruct((N, DW), jnp.int32),
        ],
    )(n2e, sel, feats, wp)


BNE = 8000
NBE = E // BNE


def _packemb_body(emb_ref, embw_ref):
    embw_ref[...] = _pack_halves(emb_ref[...], EDW)


def _packemb(emb):
    return pl.pallas_call(
        _packemb_body,
        grid=(NBE,),
        in_specs=[pl.BlockSpec((BNE, ED), lambda i: (i, 0))],
        out_specs=pl.BlockSpec((BNE, EDW), lambda i: (i, 0)),
        out_shape=jax.ShapeDtypeStruct((E, EDW), jnp.int32),
    )(emb)


def _wc_body(wp_ref, we0b_ref, wc_ref):
    wc_ref[...] = 0.5 * jnp.dot(wp_ref[...], we0b_ref[...],
                                preferred_element_type=jnp.float32)


def _wc(wp, we0b):
    return pl.pallas_call(
        _wc_body,
        out_shape=jax.ShapeDtypeStruct((D, D), jnp.float32),
    )(wp, we0b)


CH = 640
NCH = NK // CH
QR = CH // 128
_NC = 2
_NS = 16
_NW = _NC * _NS


def _gather_body(idx1_hbm, adj_hbm, featsw_hbm, embw_hbm,
                 ga_hbm, gb_hbm, gemb_hbm,
                 idx_v, av_v, bv_v, an_v, bn_v, rows_v, emb_v, sem):
    wid = lax.axis_index("s") * _NC + lax.axis_index("c")
    nt = (NCH - wid + _NW - 1) // _NW

    def body(t, carry):
        c = wid + t * _NW
        base = c * CH
        pltpu.sync_copy(idx1_hbm.at[pl.ds(base, CH)], idx_v)
        for j in range(CH // 16):
            v = idx_v[pl.ds(j * 16, 16)]
            av_v[pl.ds(j * 16, 16)] = v + v
            bv_v[pl.ds(j * 16, 16)] = v + v + 1
        cps = [pltpu.async_copy(adj_hbm.at[av_v.at[pl.ds(q * 128, 128)]],
                                an_v.at[pl.ds(q * 128, 128)], sem)
               for q in range(QR)]
        cps += [pltpu.async_copy(adj_hbm.at[bv_v.at[pl.ds(q * 128, 128)]],
                                 bn_v.at[pl.ds(q * 128, 128)], sem)
                for q in range(QR)]
        cps += [pltpu.async_copy(embw_hbm.at[idx_v.at[pl.ds(q * 128, 128)]],
                                 emb_v.at[pl.ds(q * 128, 128)], sem)
                for q in range(QR)]
        for cp in cps:
            cp.wait()
        pltpu.sync_copy(emb_v, gemb_hbm.at[pl.ds(base, CH)])
        cps = [pltpu.async_copy(featsw_hbm.at[an_v.at[pl.ds(q * 128, 128)]],
                                rows_v.at[pl.ds(q * 128, 128)], sem)
               for q in range(QR)]
        for cp in cps:
            cp.wait()
        pltpu.sync_copy(rows_v, ga_hbm.at[pl.ds(base, CH)])
        cps = [pltpu.async_copy(featsw_hbm.at[bn_v.at[pl.ds(q * 128, 128)]],
                                rows_v.at[pl.ds(q * 128, 128)], sem)
               for q in range(QR)]
        for cp in cps:
            cp.wait()
        pltpu.sync_copy(rows_v, gb_hbm.at[pl.ds(base, CH)])
        return carry

    lax.fori_loop(0, nt, body, 0)


def _gather_sc(idx1, adjf, featsw, embw):
    mesh = plsc.VectorSubcoreMesh(core_axis_name="c", subcore_axis_name="s")
    return pl.kernel(
        _gather_body,
        mesh=mesh,
        compiler_params=pltpu.CompilerParams(use_tc_tiling_on_sc=False),
        out_type=(
            jax.ShapeDtypeStruct((NK, DW), jnp.int32),
            jax.ShapeDtypeStruct((NK, DW), jnp.int32),
            jax.ShapeDtypeStruct((NK, EDW), jnp.int32),
        ),
        scratch_types=[
            pltpu.VMEM((CH,), jnp.int32),
            pltpu.VMEM((CH,), jnp.int32),
            pltpu.VMEM((CH,), jnp.int32),
            pltpu.VMEM((CH,), jnp.int32),
            pltpu.VMEM((CH,), jnp.int32),
            pltpu.VMEM((CH, DW), jnp.int32),
            pltpu.VMEM((CH, EDW), jnp.int32),
            pltpu.SemaphoreType.DMA,
        ],
    )(idx1, adjf, featsw, embw)


BN3 = 400
BS3 = BN3 * K
NB3 = N // BN3


def _heads_body(ga_ref, gb_ref, gemb_ref, f0_ref, wep_ref, we0a_ref, wc_ref,
                wn0a_ref, wn0b_ref, wn1a_ref, wn1b_ref, out_ref):
    ssum = _unpack_halves(ga_ref[...]) + _unpack_halves(gb_ref[...])
    e0 = jnp.dot(_unpack_halves(gemb_ref[...]), wep_ref[...],
                 preferred_element_type=jnp.float32)
    eo = jnp.maximum(
        jnp.dot(e0, we0a_ref[...], preferred_element_type=jnp.float32)
        + jnp.dot(ssum, wc_ref[...], preferred_element_type=jnp.float32),
        0.0)
    em0 = jnp.sum(e0.reshape(BN3, K, D), axis=1) * (1.0 / K)
    em1 = jnp.sum(eo.reshape(BN3, K, D), axis=1) * (1.0 / K)
    f0 = f0_ref[...]
    f1 = jnp.maximum(
        jnp.dot(f0, wn0a_ref[...], preferred_element_type=jnp.float32)
        + jnp.dot(em0, wn0b_ref[...], preferred_element_type=jnp.float32),
        0.0)
    f2 = jnp.maximum(
        jnp.dot(f1, wn1a_ref[...], preferred_element_type=jnp.float32)
        + jnp.dot(em1, wn1b_ref[...], preferred_element_type=jnp.float32),
        0.0)
    out_ref[...] = jnp.concatenate([f1, f2], axis=-1)[None]


def _heads(ga, gb, gemb, f0, wep, we0a, wc, wn0a, wn0b, wn1a, wn1b):
    return pl.pallas_call(
        _heads_body,
        grid=(NB3,),
        in_specs=[
            pl.BlockSpec((BS3, DW), lambda i: (i, 0)),
            pl.BlockSpec((BS3, DW), lambda i: (i, 0)),
            pl.BlockSpec((BS3, EDW), lambda i: (i, 0)),
            pl.BlockSpec((BN3, D), lambda i: (i, 0)),
            pl.BlockSpec((ED, D), lambda i: (0, 0)),
            pl.BlockSpec((D, D), lambda i: (0, 0)),
            pl.BlockSpec((D, D), lambda i: (0, 0)),
            pl.BlockSpec((D, D), lambda i: (0, 0)),
            pl.BlockSpec((D, D), lambda i: (0, 0)),
            pl.BlockSpec((D, D), lambda i: (0, 0)),
            pl.BlockSpec((D, D), lambda i: (0, 0)),
        ],
        out_specs=pl.BlockSpec((1, BN3, 2 * D), lambda i: (0, i, 0)),
        out_shape=jax.ShapeDtypeStruct((1, N, 2 * D), jnp.float32),
    )(ga, gb, gemb, f0, wep, we0a, wc, wn0a, wn0b, wn1a, wn1b)


def kernel(feats, node2edge_idx, edge_emb, edge_node_adj, sel, W_prep,
           W_edge_prep, W_e0, W_n0, W_e1, W_n1):
    del W_e1
    idx, f0, featsw = _idx_f0(node2edge_idx, sel, feats, W_prep)
    wc = _wc(W_prep, W_e0[D:])
    embw = _packemb(edge_emb)
    idx1 = idx.reshape(NK)
    adjf = edge_node_adj.reshape(2 * E)
    gaw, gbw, gembw = _gather_sc(idx1, adjf, featsw, embw)
    return _heads(gaw, gbw, gembw, f0, W_edge_prep, W_e0[:D], wc,
                  W_n0[:D], W_n0[D:], W_n1[:D], W_n1[D:])


# speedup vs baseline: 2.4353x; 1.1358x over previous
import jax
import jax.numpy as jnp
from jax import lax
from jax.experimental import pallas as pl
from jax.experimental.pallas import tpu as pltpu
from jax.experimental.pallas import tpu_sc as plsc

N = 50000
S = 16
E = N * S // 2
D = 128
ED = 16
K = 8
NK = N * K
DW = D // 2


def _pack64(f):
    lo = lax.bitcast_convert_type(f[:, :DW].astype(jnp.bfloat16), jnp.uint16)
    hi = lax.bitcast_convert_type(f[:, DW:].astype(jnp.bfloat16), jnp.uint16)
    return lo.astype(jnp.int32) | jnp.left_shift(hi.astype(jnp.int32), 16)


def _pack_pair_rows(f):
    f3 = f.reshape(f.shape[0] // 2, 2, D)
    return jnp.concatenate([_pack64(f3[:, 0, :]), _pack64(f3[:, 1, :])],
                           axis=-1)


def _lo(x):
    return lax.bitcast_convert_type(jnp.left_shift(x, 16), jnp.float32)


def _hi(x):
    return lax.bitcast_convert_type(x & jnp.int32(-65536), jnp.float32)


BN1 = 2000
NB1 = N // BN1


def _f0_body(feats_ref, wp_ref, f0_ref, fw_ref):
    f = feats_ref[...]
    f0_ref[...] = jnp.dot(f, wp_ref[...], preferred_element_type=jnp.float32)
    fw_ref[...] = _pack_pair_rows(f)


def _f0(feats, wp):
    return pl.pallas_call(
        _f0_body,
        grid=(NB1,),
        in_specs=[
            pl.BlockSpec((BN1, D), lambda i: (i, 0)),
            pl.BlockSpec((D, D), lambda i: (0, 0)),
        ],
        out_specs=[
            pl.BlockSpec((BN1, D), lambda i: (i, 0)),
            pl.BlockSpec((BN1 // 2, D), lambda i: (i, 0)),
        ],
        out_shape=[
            jax.ShapeDtypeStruct((N, D), jnp.float32),
            jax.ShapeDtypeStruct((N // 2, D), jnp.int32),
        ],
    )(feats, wp)


BNE = 4000
NBE = E // BNE


def _e0_body(emb_ref, wep_ref, ew_ref):
    e0 = jnp.dot(emb_ref[...], wep_ref[...],
                 preferred_element_type=jnp.float32)
    ew_ref[...] = _pack_pair_rows(e0)


def _e0(emb, wep):
    return pl.pallas_call(
        _e0_body,
        grid=(NBE,),
        in_specs=[
            pl.BlockSpec((BNE, ED), lambda i: (i, 0)),
            pl.BlockSpec((ED, D), lambda i: (0, 0)),
        ],
        out_specs=pl.BlockSpec((BNE // 2, D), lambda i: (i, 0)),
        out_shape=jax.ShapeDtypeStruct((E // 2, D), jnp.int32),
    )(emb, wep)


def _wc_body(wp_ref, we0b_ref, wc_ref):
    wc_ref[...] = 0.5 * jnp.dot(wp_ref[...], we0b_ref[...],
                                preferred_element_type=jnp.float32)


def _wc(wp, we0b):
    return pl.pallas_call(
        _wc_body,
        out_shape=jax.ShapeDtypeStruct((D, D), jnp.float32),
    )(wp, we0b)


CH = 640
NCH = NK // CH
QR = CH // 128
_NC = 2
_NS = 16
_NW = _NC * _NS


def _gather_body(sel_hbm, n2e_hbm, adj_hbm, fw_hbm, ew_hbm,
                 ga_hbm, gb_hbm, ge_hbm,
                 sel_v, fl_v, idx_v, av_v, bv_v,
                 rows_v, erows_v, sem_el, sem_row):
    wid = lax.axis_index("s") * _NC + lax.axis_index("c")
    nt = (NCH - wid + _NW - 1) // _NW

    def body(t, carry):
        c = wid + t * _NW
        base = c * CH
        pltpu.sync_copy(sel_hbm.at[pl.ds(base, CH)], sel_v)
        for j in range(CH // 16):
            it = lax.iota(jnp.int32, 16) + (base + j * 16)
            fl_v[pl.ds(j * 16, 16)] = (
                jnp.left_shift(lax.shift_right_logical(it, 3), 4)
                + sel_v[pl.ds(j * 16, 16)])
        cps = [pltpu.async_copy(n2e_hbm.at[fl_v.at[pl.ds(q * 128, 128)]],
                                idx_v.at[pl.ds(q * 128, 128)], sem_el)
               for q in range(QR)]
        for cp in cps:
            cp.wait()
        ecps = [pltpu.async_copy(ew_hbm.at[idx_v.at[pl.ds(q * 128, 128)]],
                                 erows_v.at[pl.ds(q * 128, 128)], sem_row)
                for q in range(QR)]
        for j in range(CH // 16):
            v = idx_v[pl.ds(j * 16, 16)]
            av_v[pl.ds(j * 16, 16)] = v + v
            bv_v[pl.ds(j * 16, 16)] = v + v + 1
        cps = [pltpu.async_copy(adj_hbm.at[av_v.at[pl.ds(q * 128, 128)]],
                                sel_v.at[pl.ds(q * 128, 128)], sem_el)
               for q in range(QR)]
        cps += [pltpu.async_copy(adj_hbm.at[bv_v.at[pl.ds(q * 128, 128)]],
                                 fl_v.at[pl.ds(q * 128, 128)], sem_el)
                for q in range(QR)]
        for cp in cps:
            cp.wait()
        cps = [pltpu.async_copy(fw_hbm.at[sel_v.at[pl.ds(q * 128, 128)]],
                                rows_v.at[pl.ds(q * 128, 128)], sem_row)
               for q in range(QR)]
        for cp in cps + ecps:
            cp.wait()
        pltpu.sync_copy(erows_v, ge_hbm.at[pl.ds(base, CH)])
        pltpu.sync_copy(rows_v, ga_hbm.at[pl.ds(base, CH)])
        cps = [pltpu.async_copy(fw_hbm.at[fl_v.at[pl.ds(q * 128, 128)]],
                                rows_v.at[pl.ds(q * 128, 128)], sem_row)
               for q in range(QR)]
        for cp in cps:
            cp.wait()
        pltpu.sync_copy(rows_v, gb_hbm.at[pl.ds(base, CH)])
        return carry

    lax.fori_loop(0, nt, body, 0)


def _gather_sc(sel1, n2e1, adjf, fw, ew):
    mesh = plsc.VectorSubcoreMesh(core_axis_name="c", subcore_axis_name="s")
    return pl.kernel(
        _gather_body,
        mesh=mesh,
        compiler_params=pltpu.CompilerParams(use_tc_tiling_on_sc=False),
        out_type=(
            jax.ShapeDtypeStruct((NK, DW), jnp.int32),
            jax.ShapeDtypeStruct((NK, DW), jnp.int32),
            jax.ShapeDtypeStruct((NK, DW), jnp.int32),
        ),
        scratch_types=[
            pltpu.VMEM((CH,), jnp.int32),
            pltpu.VMEM((CH,), jnp.int32),
            pltpu.VMEM((CH,), jnp.int32),
            pltpu.VMEM((CH,), jnp.int32),
            pltpu.VMEM((CH,), jnp.int32),
            pltpu.VMEM((CH, DW), jnp.int32),
            pltpu.VMEM((CH, DW), jnp.int32),
            pltpu.SemaphoreType.DMA,
            pltpu.SemaphoreType.DMA,
        ],
    )(sel1, n2e1, adjf, fw, ew)


BN3 = 400
BR3 = BN3 * K // 2
NB3 = N // BN3


def _heads_body(ga_ref, gb_ref, ge_ref, f0_ref, we0a_ref, wc_ref,
                wn0a_ref, wn0b_ref, wn1a_ref, wn1b_ref, out_ref):
    xa = ga_ref[...]
    xb = gb_ref[...]
    xe = ge_ref[...]
    sl = _lo(xa) + _lo(xb)
    sh = _hi(xa) + _hi(xb)
    el = _lo(xe)
    eh = _hi(xe)
    we0a = we0a_ref[...]
    wc = wc_ref[...]

    def edge_head(cols):
        return jnp.maximum(
            jnp.dot(el[:, cols], we0a[:DW], preferred_element_type=jnp.float32)
            + jnp.dot(eh[:, cols], we0a[DW:],
                      preferred_element_type=jnp.float32)
            + jnp.dot(sl[:, cols], wc[:DW], preferred_element_type=jnp.float32)
            + jnp.dot(sh[:, cols], wc[DW:],
                      preferred_element_type=jnp.float32),
            0.0)

    eo_e = edge_head(slice(0, DW))
    eo_o = edge_head(slice(DW, D))

    def s4(x):
        return jnp.sum(x.reshape(BN3, 4, x.shape[-1]), axis=1)

    em1 = (s4(eo_e) + s4(eo_o)) * (1.0 / K)
    em0l = (s4(el[:, :DW]) + s4(el[:, DW:])) * (1.0 / K)
    em0h = (s4(eh[:, :DW]) + s4(eh[:, DW:])) * (1.0 / K)
    f0 = f0_ref[...]
    f1 = jnp.maximum(
        jnp.dot(f0, wn0a_ref[...], preferred_element_type=jnp.float32)
        + jnp.dot(em0l, wn0b_ref[:DW], preferred_element_type=jnp.float32)
        + jnp.dot(em0h, wn0b_ref[DW:], preferred_element_type=jnp.float32),
        0.0)
    f2 = jnp.maximum(
        jnp.dot(f1, wn1a_ref[...], preferred_element_type=jnp.float32)
        + jnp.dot(em1, wn1b_ref[...], preferred_element_type=jnp.float32),
        0.0)
    out_ref[...] = jnp.concatenate([f1, f2], axis=-1)[None]


def _heads(ga2, gb2, ge2, f0, we0a, wc, wn0a, wn0b, wn1a, wn1b):
    return pl.pallas_call(
        _heads_body,
        grid=(NB3,),
        in_specs=[
            pl.BlockSpec((BR3, D), lambda i: (i, 0)),
            pl.BlockSpec((BR3, D), lambda i: (i, 0)),
            pl.BlockSpec((BR3, D), lambda i: (i, 0)),
            pl.BlockSpec((BN3, D), lambda i: (i, 0)),
            pl.BlockSpec((D, D), lambda i: (0, 0)),
            pl.BlockSpec((D, D), lambda i: (0, 0)),
            pl.BlockSpec((D, D), lambda i: (0, 0)),
            pl.BlockSpec((D, D), lambda i: (0, 0)),
            pl.BlockSpec((D, D), lambda i: (0, 0)),
            pl.BlockSpec((D, D), lambda i: (0, 0)),
        ],
        out_specs=pl.BlockSpec((1, BN3, 2 * D), lambda i: (0, i, 0)),
        out_shape=jax.ShapeDtypeStruct((1, N, 2 * D), jnp.float32),
    )(ga2, gb2, ge2, f0, we0a, wc, wn0a, wn0b, wn1a, wn1b)


def kernel(feats, node2edge_idx, edge_emb, edge_node_adj, sel, W_prep,
           W_edge_prep, W_e0, W_n0, W_e1, W_n1):
    del W_e1
    f0, fw128 = _f0(feats, W_prep)
    ew128 = _e0(edge_emb, W_edge_prep)
    wc = _wc(W_prep, W_e0[D:])
    sel1 = sel.reshape(NK)
    n2e1 = node2edge_idx.reshape(N * S)
    adjf = edge_node_adj.reshape(2 * E)
    fw = fw128.reshape(N, DW)
    ew = ew128.reshape(E, DW)
    gaw, gbw, gew = _gather_sc(sel1, n2e1, adjf, fw, ew)
    ga2 = gaw.reshape(NK // 2, D)
    gb2 = gbw.reshape(NK // 2, D)
    ge2 = gew.reshape(NK // 2, D)
    return _heads(ga2, gb2, ge2, f0, W_e0[:D], wc,
                  W_n0[:D], W_n0[D:], W_n1[:D], W_n1[D:])


# speedup vs baseline: 3.3296x; 1.3672x over previous
import jax
import jax.numpy as jnp
from jax import lax
from jax.experimental import pallas as pl
from jax.experimental.pallas import tpu as pltpu
from jax.experimental.pallas import tpu_sc as plsc

N = 50000
S = 16
E = N * S // 2
D = 128
ED = 16
K = 8
NK = N * K
DW = D // 2


def _pack64(f):
    lo = lax.bitcast_convert_type(f[:, :DW].astype(jnp.bfloat16), jnp.uint16)
    hi = lax.bitcast_convert_type(f[:, DW:].astype(jnp.bfloat16), jnp.uint16)
    return lo.astype(jnp.int32) | jnp.left_shift(hi.astype(jnp.int32), 16)


def _pack_pair_rows(f):
    f4 = f.reshape(f.shape[0] // 2, 2 * D)
    return jnp.concatenate([_pack64(f4[:, :D]), _pack64(f4[:, D:])],
                           axis=-1)


def _lo(x):
    return lax.bitcast_convert_type(jnp.left_shift(x, 16), jnp.float32)


def _hi(x):
    return lax.bitcast_convert_type(x & jnp.int32(-65536), jnp.float32)


BN1 = 2000
NB1 = N // BN1


def _f0_body(feats_ref, wp_ref, f0_ref, fw_ref):
    f = feats_ref[...]
    f0_ref[...] = jnp.dot(f, wp_ref[...], preferred_element_type=jnp.float32)
    fw_ref[...] = _pack_pair_rows(f)


def _f0(feats, wp):
    return pl.pallas_call(
        _f0_body,
        grid=(NB1,),
        in_specs=[
            pl.BlockSpec((BN1, D), lambda i: (i, 0)),
            pl.BlockSpec((D, D), lambda i: (0, 0)),
        ],
        out_specs=[
            pl.BlockSpec((BN1, D), lambda i: (i, 0)),
            pl.BlockSpec((BN1 // 2, D), lambda i: (i, 0)),
        ],
        out_shape=[
            jax.ShapeDtypeStruct((N, D), jnp.float32),
            jax.ShapeDtypeStruct((N // 2, D), jnp.int32),
        ],
    )(feats, wp)


BNE = 4000
NBE = E // BNE


def _e0_body(emb_ref, wep_ref, ew_ref):
    e0 = jnp.dot(emb_ref[...], wep_ref[...],
                 preferred_element_type=jnp.float32)
    ew_ref[...] = _pack_pair_rows(e0)


def _e0(emb, wep):
    return pl.pallas_call(
        _e0_body,
        grid=(NBE,),
        in_specs=[
            pl.BlockSpec((BNE, ED), lambda i: (i, 0)),
            pl.BlockSpec((ED, D), lambda i: (0, 0)),
        ],
        out_specs=pl.BlockSpec((BNE // 2, D), lambda i: (i, 0)),
        out_shape=jax.ShapeDtypeStruct((E // 2, D), jnp.int32),
    )(emb, wep)


def _wc_body(wp_ref, we0b_ref, wc_ref):
    wc_ref[...] = 0.5 * jnp.dot(wp_ref[...], we0b_ref[...],
                                preferred_element_type=jnp.float32)


def _wc(wp, we0b):
    return pl.pallas_call(
        _wc_body,
        out_shape=jax.ShapeDtypeStruct((D, D), jnp.float32),
    )(wp, we0b)


CH = 640
NCH = NK // CH
QR = CH // 128
_NC = 2
_NS = 16
_NW = _NC * _NS


def _gather_body(sel_hbm, n2e_hbm, adj_hbm, fw_hbm, ew_hbm,
                 ga_hbm, gb_hbm, ge_hbm,
                 sel_v, fl_v, idx_v, av_v, bv_v,
                 rows_v, erows_v, sem_el, sem_row):
    wid = lax.axis_index("s") * _NC + lax.axis_index("c")
    nt = (NCH - wid + _NW - 1) // _NW

    def body(t, carry):
        c = wid + t * _NW
        base = c * CH
        pltpu.sync_copy(sel_hbm.at[pl.ds(base, CH)], sel_v)
        for j in range(CH // 16):
            it = lax.iota(jnp.int32, 16) + (base + j * 16)
            fl_v[pl.ds(j * 16, 16)] = (
                jnp.left_shift(lax.shift_right_logical(it, 3), 4)
                + sel_v[pl.ds(j * 16, 16)])
        cps = [pltpu.async_copy(n2e_hbm.at[fl_v.at[pl.ds(q * 128, 128)]],
                                idx_v.at[pl.ds(q * 128, 128)], sem_el)
               for q in range(QR)]
        for cp in cps:
            cp.wait()
        ecps = [pltpu.async_copy(ew_hbm.at[idx_v.at[pl.ds(q * 128, 128)]],
                                 erows_v.at[pl.ds(q * 128, 128)], sem_row)
                for q in range(QR)]
        for j in range(CH // 16):
            v = idx_v[pl.ds(j * 16, 16)]
            av_v[pl.ds(j * 16, 16)] = v + v
            bv_v[pl.ds(j * 16, 16)] = v + v + 1
        cps = [pltpu.async_copy(adj_hbm.at[av_v.at[pl.ds(q * 128, 128)]],
                                sel_v.at[pl.ds(q * 128, 128)], sem_el)
               for q in range(QR)]
        cps += [pltpu.async_copy(adj_hbm.at[bv_v.at[pl.ds(q * 128, 128)]],
                                 fl_v.at[pl.ds(q * 128, 128)], sem_el)
                for q in range(QR)]
        for cp in cps:
            cp.wait()
        cps = [pltpu.async_copy(fw_hbm.at[sel_v.at[pl.ds(q * 128, 128)]],
                                rows_v.at[pl.ds(q * 128, 128)], sem_row)
               for q in range(QR)]
        for cp in cps + ecps:
            cp.wait()
        pltpu.sync_copy(erows_v, ge_hbm.at[pl.ds(base, CH)])
        pltpu.sync_copy(rows_v, ga_hbm.at[pl.ds(base, CH)])
        cps = [pltpu.async_copy(fw_hbm.at[fl_v.at[pl.ds(q * 128, 128)]],
                                rows_v.at[pl.ds(q * 128, 128)], sem_row)
               for q in range(QR)]
        for cp in cps:
            cp.wait()
        pltpu.sync_copy(rows_v, gb_hbm.at[pl.ds(base, CH)])
        return carry

    lax.fori_loop(0, nt, body, 0)


def _gather_sc(sel1, n2e1, adjf, fw, ew):
    mesh = plsc.VectorSubcoreMesh(core_axis_name="c", subcore_axis_name="s")
    return pl.kernel(
        _gather_body,
        mesh=mesh,
        compiler_params=pltpu.CompilerParams(use_tc_tiling_on_sc=False),
        out_type=(
            jax.ShapeDtypeStruct((NK, DW), jnp.int32),
            jax.ShapeDtypeStruct((NK, DW), jnp.int32),
            jax.ShapeDtypeStruct((NK, DW), jnp.int32),
        ),
        scratch_types=[
            pltpu.VMEM((CH,), jnp.int32),
            pltpu.VMEM((CH,), jnp.int32),
            pltpu.VMEM((CH,), jnp.int32),
            pltpu.VMEM((CH,), jnp.int32),
            pltpu.VMEM((CH,), jnp.int32),
            pltpu.VMEM((CH, DW), jnp.int32),
            pltpu.VMEM((CH, DW), jnp.int32),
            pltpu.SemaphoreType.DMA,
            pltpu.SemaphoreType.DMA,
        ],
    )(sel1, n2e1, adjf, fw, ew)


BN3 = 400
BR3 = BN3 * K // 2
NB3 = N // BN3


def _heads_body(ga_ref, gb_ref, ge_ref, f0_ref, we0a_ref, wc_ref,
                wn0a_ref, wn0b_ref, wn1a_ref, wn1b_ref, out_ref):
    xa = ga_ref[...]
    xb = gb_ref[...]
    xe = ge_ref[...]
    sl = _lo(xa) + _lo(xb)
    sh = _hi(xa) + _hi(xb)
    el = _lo(xe)
    eh = _hi(xe)
    we0a = we0a_ref[...]
    wc = wc_ref[...]

    def edge_head(cols):
        return jnp.maximum(
            jnp.dot(el[:, cols], we0a[:DW], preferred_element_type=jnp.float32)
            + jnp.dot(eh[:, cols], we0a[DW:],
                      preferred_element_type=jnp.float32)
            + jnp.dot(sl[:, cols], wc[:DW], preferred_element_type=jnp.float32)
            + jnp.dot(sh[:, cols], wc[DW:],
                      preferred_element_type=jnp.float32),
            0.0)

    eo_e = edge_head(slice(0, DW))
    eo_o = edge_head(slice(DW, D))

    def s4(x):
        x4 = x.reshape(BN3, 4 * D)
        return (x4[:, :D] + x4[:, D:2 * D] + x4[:, 2 * D:3 * D]
                + x4[:, 3 * D:])

    em1 = (s4(eo_e) + s4(eo_o)) * (1.0 / K)
    el4 = s4(el)
    eh4 = s4(eh)
    em0l = (el4[:, :DW] + el4[:, DW:]) * (1.0 / K)
    em0h = (eh4[:, :DW] + eh4[:, DW:]) * (1.0 / K)
    f0 = f0_ref[...]
    f1 = jnp.maximum(
        jnp.dot(f0, wn0a_ref[...], preferred_element_type=jnp.float32)
        + jnp.dot(em0l, wn0b_ref[:DW], preferred_element_type=jnp.float32)
        + jnp.dot(em0h, wn0b_ref[DW:], preferred_element_type=jnp.float32),
        0.0)
    f2 = jnp.maximum(
        jnp.dot(f1, wn1a_ref[...], preferred_element_type=jnp.float32)
        + jnp.dot(em1, wn1b_ref[...], preferred_element_type=jnp.float32),
        0.0)
    out_ref[...] = jnp.concatenate([f1, f2], axis=-1)[None]


def _heads(ga2, gb2, ge2, f0, we0a, wc, wn0a, wn0b, wn1a, wn1b):
    return pl.pallas_call(
        _heads_body,
        grid=(NB3,),
        in_specs=[
            pl.BlockSpec((BR3, D), lambda i: (i, 0)),
            pl.BlockSpec((BR3, D), lambda i: (i, 0)),
            pl.BlockSpec((BR3, D), lambda i: (i, 0)),
            pl.BlockSpec((BN3, D), lambda i: (i, 0)),
            pl.BlockSpec((D, D), lambda i: (0, 0)),
            pl.BlockSpec((D, D), lambda i: (0, 0)),
            pl.BlockSpec((D, D), lambda i: (0, 0)),
            pl.BlockSpec((D, D), lambda i: (0, 0)),
            pl.BlockSpec((D, D), lambda i: (0, 0)),
            pl.BlockSpec((D, D), lambda i: (0, 0)),
        ],
        out_specs=pl.BlockSpec((1, BN3, 2 * D), lambda i: (0, i, 0)),
        out_shape=jax.ShapeDtypeStruct((1, N, 2 * D), jnp.float32),
    )(ga2, gb2, ge2, f0, we0a, wc, wn0a, wn0b, wn1a, wn1b)


def kernel(feats, node2edge_idx, edge_emb, edge_node_adj, sel, W_prep,
           W_edge_prep, W_e0, W_n0, W_e1, W_n1):
    del W_e1
    f0, fw128 = _f0(feats, W_prep)
    ew128 = _e0(edge_emb, W_edge_prep)
    wc = _wc(W_prep, W_e0[D:])
    sel1 = sel.reshape(NK)
    n2e1 = node2edge_idx.reshape(N * S)
    adjf = edge_node_adj.reshape(2 * E)
    fw = fw128.reshape(N, DW)
    ew = ew128.reshape(E, DW)
    gaw, gbw, gew = _gather_sc(sel1, n2e1, adjf, fw, ew)
    ga2 = gaw.reshape(NK // 2, D)
    gb2 = gbw.reshape(NK // 2, D)
    ge2 = gew.reshape(NK // 2, D)
    return _heads(ga2, gb2, ge2, f0, W_e0[:D], wc,
                  W_n0[:D], W_n0[D:], W_n1[:D], W_n1[D:])


# speedup vs baseline: 4.8003x; 1.4417x over previous
import jax
import jax.numpy as jnp
from jax import lax
from jax.experimental import pallas as pl
from jax.experimental.pallas import tpu as pltpu
from jax.experimental.pallas import tpu_sc as plsc

N = 50000
S = 16
E = N * S // 2
D = 128
ED = 16
K = 8
NK = N * K
DW = D // 2


def _pack64(f):
    lo = lax.bitcast_convert_type(f[:, :DW].astype(jnp.bfloat16), jnp.uint16)
    hi = lax.bitcast_convert_type(f[:, DW:].astype(jnp.bfloat16), jnp.uint16)
    return lo.astype(jnp.int32) | jnp.left_shift(hi.astype(jnp.int32), 16)


def _pack_pair_rows(f):
    f4 = f.reshape(f.shape[0] // 2, 2 * D)
    return jnp.concatenate([_pack64(f4[:, :D]), _pack64(f4[:, D:])],
                           axis=-1)


def _lo(x):
    return lax.bitcast_convert_type(jnp.left_shift(x, 16), jnp.float32)


def _hi(x):
    return lax.bitcast_convert_type(x & jnp.int32(-65536), jnp.float32)


BN1 = 2000
NB1 = N // BN1


def _f0_body(feats_ref, wp_ref, f0_ref, fw_ref):
    f = feats_ref[...]
    f0_ref[...] = jnp.dot(f, wp_ref[...], preferred_element_type=jnp.float32)
    fw_ref[...] = _pack_pair_rows(f)


def _f0(feats, wp):
    return pl.pallas_call(
        _f0_body,
        grid=(NB1,),
        in_specs=[
            pl.BlockSpec((BN1, D), lambda i: (i, 0)),
            pl.BlockSpec((D, D), lambda i: (0, 0)),
        ],
        out_specs=[
            pl.BlockSpec((BN1, D), lambda i: (i, 0)),
            pl.BlockSpec((BN1 // 2, D), lambda i: (i, 0)),
        ],
        out_shape=[
            jax.ShapeDtypeStruct((N, D), jnp.float32),
            jax.ShapeDtypeStruct((N // 2, D), jnp.int32),
        ],
    )(feats, wp)


BNE = 3200
NBE = E // BNE


def _e0_body(embt_ref, wep_ref, ew_ref):
    e0 = lax.dot_general(embt_ref[...], wep_ref[...],
                         (((0,), (0,)), ((), ())),
                         preferred_element_type=jnp.float32)
    ew_ref[...] = _pack_pair_rows(e0)


def _e0(embt, wep):
    return pl.pallas_call(
        _e0_body,
        grid=(NBE,),
        in_specs=[
            pl.BlockSpec((ED, BNE), lambda i: (0, i)),
            pl.BlockSpec((ED, D), lambda i: (0, 0)),
        ],
        out_specs=pl.BlockSpec((BNE // 2, D), lambda i: (i, 0)),
        out_shape=jax.ShapeDtypeStruct((E // 2, D), jnp.int32),
    )(embt, wep)


def _wc_body(wp_ref, we0b_ref, wc_ref):
    wc_ref[...] = 0.5 * jnp.dot(wp_ref[...], we0b_ref[...],
                                preferred_element_type=jnp.float32)


def _wc(wp, we0b):
    return pl.pallas_call(
        _wc_body,
        out_shape=jax.ShapeDtypeStruct((D, D), jnp.float32),
    )(wp, we0b)


CH = 640
NCH = NK // CH
QR = CH // 128
_NC = 2
_NS = 16
_NW = _NC * _NS


def _gather_body(sel_hbm, n2et_hbm, adja_hbm, adjb_hbm, fw_hbm, ew_hbm,
                 ga_hbm, gb_hbm, ge_hbm,
                 sel_v, fl_v, idx_v, an_v, bn_v,
                 rows_v, erows_v, sem_el, sem_row):
    wid = lax.axis_index("s") * _NC + lax.axis_index("c")
    nt = (NCH - wid + _NW - 1) // _NW

    def body(t, carry):
        c = wid + t * _NW
        base = c * CH
        pltpu.sync_copy(sel_hbm.at[pl.ds(base, CH)], sel_v)
        for j in range(CH // 16):
            it = lax.iota(jnp.int32, 16) + (base + j * 16)
            fl_v[pl.ds(j * 16, 16)] = (
                sel_v[pl.ds(j * 16, 16)] * N
                + lax.shift_right_logical(it, 3))
        cps = [pltpu.async_copy(n2et_hbm.at[fl_v.at[pl.ds(q * 128, 128)]],
                                idx_v.at[pl.ds(q * 128, 128)], sem_el)
               for q in range(QR)]
        for cp in cps:
            cp.wait()
        ecps = [pltpu.async_copy(ew_hbm.at[idx_v.at[pl.ds(q * 128, 128)]],
                                 erows_v.at[pl.ds(q * 128, 128)], sem_row)
                for q in range(QR)]
        cps = [pltpu.async_copy(adja_hbm.at[idx_v.at[pl.ds(q * 128, 128)]],
                                an_v.at[pl.ds(q * 128, 128)], sem_el)
               for q in range(QR)]
        cps += [pltpu.async_copy(adjb_hbm.at[idx_v.at[pl.ds(q * 128, 128)]],
                                 bn_v.at[pl.ds(q * 128, 128)], sem_el)
                for q in range(QR)]
        for cp in cps:
            cp.wait()
        cps = [pltpu.async_copy(fw_hbm.at[an_v.at[pl.ds(q * 128, 128)]],
                                rows_v.at[pl.ds(q * 128, 128)], sem_row)
               for q in range(QR)]
        for cp in cps + ecps:
            cp.wait()
        pltpu.sync_copy(erows_v, ge_hbm.at[pl.ds(base, CH)])
        pltpu.sync_copy(rows_v, ga_hbm.at[pl.ds(base, CH)])
        cps = [pltpu.async_copy(fw_hbm.at[bn_v.at[pl.ds(q * 128, 128)]],
                                rows_v.at[pl.ds(q * 128, 128)], sem_row)
               for q in range(QR)]
        for cp in cps:
            cp.wait()
        pltpu.sync_copy(rows_v, gb_hbm.at[pl.ds(base, CH)])
        return carry

    lax.fori_loop(0, nt, body, 0)


def _gather_sc(sel1, n2et1, adja, adjb, fw, ew):
    mesh = plsc.VectorSubcoreMesh(core_axis_name="c", subcore_axis_name="s")
    return pl.kernel(
        _gather_body,
        mesh=mesh,
        compiler_params=pltpu.CompilerParams(use_tc_tiling_on_sc=False),
        out_type=(
            jax.ShapeDtypeStruct((NK, DW), jnp.int32),
            jax.ShapeDtypeStruct((NK, DW), jnp.int32),
            jax.ShapeDtypeStruct((NK, DW), jnp.int32),
        ),
        scratch_types=[
            pltpu.VMEM((CH,), jnp.int32),
            pltpu.VMEM((CH,), jnp.int32),
            pltpu.VMEM((CH,), jnp.int32),
            pltpu.VMEM((CH,), jnp.int32),
            pltpu.VMEM((CH,), jnp.int32),
            pltpu.VMEM((CH, DW), jnp.int32),
            pltpu.VMEM((CH, DW), jnp.int32),
            pltpu.SemaphoreType.DMA,
            pltpu.SemaphoreType.DMA,
        ],
    )(sel1, n2et1, adja, adjb, fw, ew)


BN3 = 400
BR3 = BN3 * K // 2
NB3 = N // BN3


def _heads_body(ga_ref, gb_ref, ge_ref, f0_ref, we0a_ref, wc_ref,
                wn0a_ref, wn0b_ref, wn1a_ref, wn1b_ref, out_ref):
    xa = ga_ref[...]
    xb = gb_ref[...]
    xe = ge_ref[...]
    sl = _lo(xa) + _lo(xb)
    sh = _hi(xa) + _hi(xb)
    el = _lo(xe)
    eh = _hi(xe)
    we0a = we0a_ref[...]
    wc = wc_ref[...]

    def edge_head(cols):
        return jnp.maximum(
            jnp.dot(el[:, cols], we0a[:DW], preferred_element_type=jnp.float32)
            + jnp.dot(eh[:, cols], we0a[DW:],
                      preferred_element_type=jnp.float32)
            + jnp.dot(sl[:, cols], wc[:DW], preferred_element_type=jnp.float32)
            + jnp.dot(sh[:, cols], wc[DW:],
                      preferred_element_type=jnp.float32),
            0.0)

    eo_e = edge_head(slice(0, DW))
    eo_o = edge_head(slice(DW, D))

    def s4(x):
        x4 = x.reshape(BN3, 4 * D)
        return (x4[:, :D] + x4[:, D:2 * D] + x4[:, 2 * D:3 * D]
                + x4[:, 3 * D:])

    em1 = (s4(eo_e) + s4(eo_o)) * (1.0 / K)
    el4 = s4(el)
    eh4 = s4(eh)
    em0l = (el4[:, :DW] + el4[:, DW:]) * (1.0 / K)
    em0h = (eh4[:, :DW] + eh4[:, DW:]) * (1.0 / K)
    f0 = f0_ref[...]
    f1 = jnp.maximum(
        jnp.dot(f0, wn0a_ref[...], preferred_element_type=jnp.float32)
        + jnp.dot(em0l, wn0b_ref[:DW], preferred_element_type=jnp.float32)
        + jnp.dot(em0h, wn0b_ref[DW:], preferred_element_type=jnp.float32),
        0.0)
    f2 = jnp.maximum(
        jnp.dot(f1, wn1a_ref[...], preferred_element_type=jnp.float32)
        + jnp.dot(em1, wn1b_ref[...], preferred_element_type=jnp.float32),
        0.0)
    out_ref[...] = jnp.concatenate([f1, f2], axis=-1)[None]


def _heads(ga2, gb2, ge2, f0, we0a, wc, wn0a, wn0b, wn1a, wn1b):
    return pl.pallas_call(
        _heads_body,
        grid=(NB3,),
        in_specs=[
            pl.BlockSpec((BR3, D), lambda i: (i, 0)),
            pl.BlockSpec((BR3, D), lambda i: (i, 0)),
            pl.BlockSpec((BR3, D), lambda i: (i, 0)),
            pl.BlockSpec((BN3, D), lambda i: (i, 0)),
            pl.BlockSpec((D, D), lambda i: (0, 0)),
            pl.BlockSpec((D, D), lambda i: (0, 0)),
            pl.BlockSpec((D, D), lambda i: (0, 0)),
            pl.BlockSpec((D, D), lambda i: (0, 0)),
            pl.BlockSpec((D, D), lambda i: (0, 0)),
            pl.BlockSpec((D, D), lambda i: (0, 0)),
        ],
        out_specs=pl.BlockSpec((1, BN3, 2 * D), lambda i: (0, i, 0)),
        out_shape=jax.ShapeDtypeStruct((1, N, 2 * D), jnp.float32),
    )(ga2, gb2, ge2, f0, we0a, wc, wn0a, wn0b, wn1a, wn1b)


def kernel(feats, node2edge_idx, edge_emb, edge_node_adj, sel, W_prep,
           W_edge_prep, W_e0, W_n0, W_e1, W_n1):
    del W_e1
    f0, fw128 = _f0(feats, W_prep)
    ew128 = _e0(edge_emb.T, W_edge_prep)
    wc = _wc(W_prep, W_e0[D:])
    sel1 = sel.reshape(NK)
    n2et1 = node2edge_idx.T.reshape(N * S)
    adjt = edge_node_adj.T
    adja = adjt[0]
    adjb = adjt[1]
    fw = fw128.reshape(N, DW)
    ew = ew128.reshape(E, DW)
    gaw, gbw, gew = _gather_sc(sel1, n2et1, adja, adjb, fw, ew)
    ga2 = gaw.reshape(NK // 2, D)
    gb2 = gbw.reshape(NK // 2, D)
    ge2 = gew.reshape(NK // 2, D)
    return _heads(ga2, gb2, ge2, f0, W_e0[:D], wc,
                  W_n0[:D], W_n0[D:], W_n1[:D], W_n1[D:])


# speedup vs baseline: 4.9105x; 1.0230x over previous
import jax
import jax.numpy as jnp
from jax import lax
from jax.experimental import pallas as pl
from jax.experimental.pallas import tpu as pltpu
from jax.experimental.pallas import tpu_sc as plsc

N = 50000
S = 16
E = N * S // 2
D = 128
ED = 16
K = 8
NK = N * K
DW = D // 2


def _pack64(f):
    lo = lax.bitcast_convert_type(f[:, :DW].astype(jnp.bfloat16), jnp.uint16)
    hi = lax.bitcast_convert_type(f[:, DW:].astype(jnp.bfloat16), jnp.uint16)
    return lo.astype(jnp.int32) | jnp.left_shift(hi.astype(jnp.int32), 16)


def _pack_pair_rows(f):
    f4 = f.reshape(f.shape[0] // 2, 2 * D)
    return jnp.concatenate([_pack64(f4[:, :D]), _pack64(f4[:, D:])],
                           axis=-1)


def _lo(x):
    return lax.bitcast_convert_type(jnp.left_shift(x, 16), jnp.float32)


def _hi(x):
    return lax.bitcast_convert_type(x & jnp.int32(-65536), jnp.float32)


BN1 = 2000
NB1 = N // BN1


def _f0_body(feats_ref, wp_ref, f0_ref, fw_ref):
    f = feats_ref[...]
    f0_ref[...] = jnp.dot(f, wp_ref[...], preferred_element_type=jnp.float32)
    fw_ref[...] = _pack_pair_rows(f)


def _f0(feats, wp):
    return pl.pallas_call(
        _f0_body,
        grid=(NB1,),
        in_specs=[
            pl.BlockSpec((BN1, D), lambda i: (i, 0)),
            pl.BlockSpec((D, D), lambda i: (0, 0)),
        ],
        out_specs=[
            pl.BlockSpec((BN1, D), lambda i: (i, 0)),
            pl.BlockSpec((BN1 // 2, D), lambda i: (i, 0)),
        ],
        out_shape=[
            jax.ShapeDtypeStruct((N, D), jnp.float32),
            jax.ShapeDtypeStruct((N // 2, D), jnp.int32),
        ],
    )(feats, wp)


BNE = 3200
NBE = E // BNE


def _e0_body(embt_ref, wep_ref, ew_ref):
    e0 = lax.dot_general(embt_ref[...], wep_ref[...],
                         (((0,), (0,)), ((), ())),
                         preferred_element_type=jnp.float32)
    ew_ref[...] = _pack_pair_rows(e0)


def _e0(embt, wep):
    return pl.pallas_call(
        _e0_body,
        grid=(NBE,),
        in_specs=[
            pl.BlockSpec((ED, BNE), lambda i: (0, i)),
            pl.BlockSpec((ED, D), lambda i: (0, 0)),
        ],
        out_specs=pl.BlockSpec((BNE // 2, D), lambda i: (i, 0)),
        out_shape=jax.ShapeDtypeStruct((E // 2, D), jnp.int32),
    )(embt, wep)


def _wc_body(wp_ref, we0b_ref, wc_ref):
    wc_ref[...] = (0.5 * jnp.dot(wp_ref[...], we0b_ref[...],
                                 preferred_element_type=jnp.float32)
                   ).astype(jnp.bfloat16)


def _wc(wp, we0b):
    return pl.pallas_call(
        _wc_body,
        out_shape=jax.ShapeDtypeStruct((D, D), jnp.bfloat16),
    )(wp, we0b)


CH = 640
NCH = NK // CH
QR = CH // 128
_NC = 2
_NS = 16
_NW = _NC * _NS


def _gather_body(sel_hbm, n2et_hbm, adja_hbm, adjb_hbm, fw_hbm, ew_hbm,
                 ga_hbm, gb_hbm, ge_hbm,
                 sel_v, fl_v, idx_v, an_v, bn_v,
                 rows_v, erows_v, sem_el, sem_row):
    wid = lax.axis_index("s") * _NC + lax.axis_index("c")
    nt = (NCH - wid + _NW - 1) // _NW

    def body(t, carry):
        c = wid + t * _NW
        base = c * CH
        pltpu.sync_copy(sel_hbm.at[pl.ds(base, CH)], sel_v)
        for j in range(CH // 16):
            it = lax.iota(jnp.int32, 16) + (base + j * 16)
            fl_v[pl.ds(j * 16, 16)] = (
                sel_v[pl.ds(j * 16, 16)] * N
                + lax.shift_right_logical(it, 3))
        cps = [pltpu.async_copy(n2et_hbm.at[fl_v.at[pl.ds(q * 128, 128)]],
                                idx_v.at[pl.ds(q * 128, 128)], sem_el)
               for q in range(QR)]
        for cp in cps:
            cp.wait()
        ecps = [pltpu.async_copy(ew_hbm.at[idx_v.at[pl.ds(q * 128, 128)]],
                                 erows_v.at[pl.ds(q * 128, 128)], sem_row)
                for q in range(QR)]
        cps = [pltpu.async_copy(adja_hbm.at[idx_v.at[pl.ds(q * 128, 128)]],
                                an_v.at[pl.ds(q * 128, 128)], sem_el)
               for q in range(QR)]
        cps += [pltpu.async_copy(adjb_hbm.at[idx_v.at[pl.ds(q * 128, 128)]],
                                 bn_v.at[pl.ds(q * 128, 128)], sem_el)
                for q in range(QR)]
        for cp in cps:
            cp.wait()
        cps = [pltpu.async_copy(fw_hbm.at[an_v.at[pl.ds(q * 128, 128)]],
                                rows_v.at[pl.ds(q * 128, 128)], sem_row)
               for q in range(QR)]
        for cp in cps + ecps:
            cp.wait()
        pltpu.sync_copy(erows_v, ge_hbm.at[pl.ds(base, CH)])
        pltpu.sync_copy(rows_v, ga_hbm.at[pl.ds(base, CH)])
        cps = [pltpu.async_copy(fw_hbm.at[bn_v.at[pl.ds(q * 128, 128)]],
                                rows_v.at[pl.ds(q * 128, 128)], sem_row)
               for q in range(QR)]
        for cp in cps:
            cp.wait()
        pltpu.sync_copy(rows_v, gb_hbm.at[pl.ds(base, CH)])
        return carry

    lax.fori_loop(0, nt, body, 0)


def _gather_sc(sel1, n2et1, adja, adjb, fw, ew):
    mesh = plsc.VectorSubcoreMesh(core_axis_name="c", subcore_axis_name="s")
    return pl.kernel(
        _gather_body,
        mesh=mesh,
        compiler_params=pltpu.CompilerParams(use_tc_tiling_on_sc=False),
        out_type=(
            jax.ShapeDtypeStruct((NK, DW), jnp.int32),
            jax.ShapeDtypeStruct((NK, DW), jnp.int32),
            jax.ShapeDtypeStruct((NK, DW), jnp.int32),
        ),
        scratch_types=[
            pltpu.VMEM((CH,), jnp.int32),
            pltpu.VMEM((CH,), jnp.int32),
            pltpu.VMEM((CH,), jnp.int32),
            pltpu.VMEM((CH,), jnp.int32),
            pltpu.VMEM((CH,), jnp.int32),
            pltpu.VMEM((CH, DW), jnp.int32),
            pltpu.VMEM((CH, DW), jnp.int32),
            pltpu.SemaphoreType.DMA,
            pltpu.SemaphoreType.DMA,
        ],
    )(sel1, n2et1, adja, adjb, fw, ew)


BN3 = 400
BR3 = BN3 * K // 2
NB3 = N // BN3


def _heads_body(ga_ref, gb_ref, ge_ref, f0_ref, we0a_ref, wc_ref,
                wn0a_ref, wn0b_ref, wn1a_ref, wn1b_ref, out_ref):
    xa = ga_ref[...]
    xb = gb_ref[...]
    xe = ge_ref[...]
    sl = _lo(xa) + _lo(xb)
    sh = _hi(xa) + _hi(xb)
    el = _lo(xe)
    eh = _hi(xe)
    slb = sl.astype(jnp.bfloat16)
    shb = sh.astype(jnp.bfloat16)
    elb = el.astype(jnp.bfloat16)
    ehb = eh.astype(jnp.bfloat16)
    we0a = we0a_ref[...]
    wc = wc_ref[...]

    def edge_head(cols):
        return jnp.maximum(
            jnp.dot(elb[:, cols], we0a[:DW],
                    preferred_element_type=jnp.float32)
            + jnp.dot(ehb[:, cols], we0a[DW:],
                      preferred_element_type=jnp.float32)
            + jnp.dot(slb[:, cols], wc[:DW],
                      preferred_element_type=jnp.float32)
            + jnp.dot(shb[:, cols], wc[DW:],
                      preferred_element_type=jnp.float32),
            0.0)

    eo_e = edge_head(slice(0, DW))
    eo_o = edge_head(slice(DW, D))

    def s4(x):
        x4 = x.reshape(BN3, 4 * D)
        return (x4[:, :D] + x4[:, D:2 * D] + x4[:, 2 * D:3 * D]
                + x4[:, 3 * D:])

    em1 = (s4(eo_e) + s4(eo_o)) * (1.0 / K)
    el4 = s4(el)
    eh4 = s4(eh)
    em0l = (el4[:, :DW] + el4[:, DW:]) * (1.0 / K)
    em0h = (eh4[:, :DW] + eh4[:, DW:]) * (1.0 / K)
    f0 = f0_ref[...]
    f1 = jnp.maximum(
        jnp.dot(f0, wn0a_ref[...], preferred_element_type=jnp.float32)
        + jnp.dot(em0l, wn0b_ref[:DW], preferred_element_type=jnp.float32)
        + jnp.dot(em0h, wn0b_ref[DW:], preferred_element_type=jnp.float32),
        0.0)
    f2 = jnp.maximum(
        jnp.dot(f1, wn1a_ref[...], preferred_element_type=jnp.float32)
        + jnp.dot(em1, wn1b_ref[...], preferred_element_type=jnp.float32),
        0.0)
    out_ref[...] = jnp.concatenate([f1, f2], axis=-1)[None]


def _heads(ga2, gb2, ge2, f0, we0a, wc, wn0a, wn0b, wn1a, wn1b):
    return pl.pallas_call(
        _heads_body,
        grid=(NB3,),
        in_specs=[
            pl.BlockSpec((BR3, D), lambda i: (i, 0)),
            pl.BlockSpec((BR3, D), lambda i: (i, 0)),
            pl.BlockSpec((BR3, D), lambda i: (i, 0)),
            pl.BlockSpec((BN3, D), lambda i: (i, 0)),
            pl.BlockSpec((D, D), lambda i: (0, 0)),
            pl.BlockSpec((D, D), lambda i: (0, 0)),
            pl.BlockSpec((D, D), lambda i: (0, 0)),
            pl.BlockSpec((D, D), lambda i: (0, 0)),
            pl.BlockSpec((D, D), lambda i: (0, 0)),
            pl.BlockSpec((D, D), lambda i: (0, 0)),
        ],
        out_specs=pl.BlockSpec((1, BN3, 2 * D), lambda i: (0, i, 0)),
        out_shape=jax.ShapeDtypeStruct((1, N, 2 * D), jnp.float32),
    )(ga2, gb2, ge2, f0, we0a, wc, wn0a, wn0b, wn1a, wn1b)


def kernel(feats, node2edge_idx, edge_emb, edge_node_adj, sel, W_prep,
           W_edge_prep, W_e0, W_n0, W_e1, W_n1):
    del W_e1
    f0, fw128 = _f0(feats, W_prep)
    ew128 = _e0(edge_emb.T, W_edge_prep)
    wc = _wc(W_prep, W_e0[D:])
    sel1 = sel.reshape(NK)
    n2et1 = node2edge_idx.T.reshape(N * S)
    adjt = edge_node_adj.T
    adja = adjt[0]
    adjb = adjt[1]
    fw = fw128.reshape(N, DW)
    ew = ew128.reshape(E, DW)
    gaw, gbw, gew = _gather_sc(sel1, n2et1, adja, adjb, fw, ew)
    ga2 = gaw.reshape(NK // 2, D)
    gb2 = gbw.reshape(NK // 2, D)
    ge2 = gew.reshape(NK // 2, D)
    return _heads(ga2, gb2, ge2, f0, W_e0[:D].astype(jnp.bfloat16), wc,
                  W_n0[:D], W_n0[D:], W_n1[:D], W_n1[D:])


# speedup vs baseline: 5.2322x; 1.0655x over previous
import jax
import jax.numpy as jnp
from jax import lax
from jax.experimental import pallas as pl
from jax.experimental.pallas import tpu as pltpu
from jax.experimental.pallas import tpu_sc as plsc

N = 50000
S = 16
E = N * S // 2
D = 128
ED = 16
K = 8
NK = N * K
DW = D // 2


def _pack64(f):
    lo = lax.bitcast_convert_type(f[:, :DW].astype(jnp.bfloat16), jnp.uint16)
    hi = lax.bitcast_convert_type(f[:, DW:].astype(jnp.bfloat16), jnp.uint16)
    return lo.astype(jnp.int32) | jnp.left_shift(hi.astype(jnp.int32), 16)


def _pack_pair_rows(f):
    f4 = f.reshape(f.shape[0] // 2, 2 * D)
    return jnp.concatenate([_pack64(f4[:, :D]), _pack64(f4[:, D:])],
                           axis=-1)


def _lo(x):
    return lax.bitcast_convert_type(jnp.left_shift(x, 16), jnp.float32)


def _hi(x):
    return lax.bitcast_convert_type(x & jnp.int32(-65536), jnp.float32)


BN1 = 2000
NB1 = N // BN1


def _f0_body(feats_ref, wp_ref, f0_ref, fw_ref):
    f = feats_ref[...]
    f0_ref[...] = jnp.dot(f, wp_ref[...], preferred_element_type=jnp.float32)
    fw_ref[...] = _pack_pair_rows(f)


def _f0(feats, wp):
    return pl.pallas_call(
        _f0_body,
        grid=(NB1,),
        in_specs=[
            pl.BlockSpec((BN1, D), lambda i: (i, 0)),
            pl.BlockSpec((D, D), lambda i: (0, 0)),
        ],
        out_specs=[
            pl.BlockSpec((BN1, D), lambda i: (i, 0)),
            pl.BlockSpec((BN1 // 2, D), lambda i: (i, 0)),
        ],
        out_shape=[
            jax.ShapeDtypeStruct((N, D), jnp.float32),
            jax.ShapeDtypeStruct((N // 2, D), jnp.int32),
        ],
    )(feats, wp)


BNE = 3200
NBE = E // BNE


def _e0_body(embt_ref, wep_ref, ew_ref):
    e0 = lax.dot_general(embt_ref[...], wep_ref[...],
                         (((0,), (0,)), ((), ())),
                         preferred_element_type=jnp.float32)
    ew_ref[...] = _pack_pair_rows(e0)


def _e0(embt, wep):
    return pl.pallas_call(
        _e0_body,
        grid=(NBE,),
        in_specs=[
            pl.BlockSpec((ED, BNE), lambda i: (0, i)),
            pl.BlockSpec((ED, D), lambda i: (0, 0)),
        ],
        out_specs=pl.BlockSpec((BNE // 2, D), lambda i: (i, 0)),
        out_shape=jax.ShapeDtypeStruct((E // 2, D), jnp.int32),
    )(embt, wep)


def _wc_body(wp_ref, we0b_ref, wc_ref):
    wc_ref[...] = (0.5 * jnp.dot(wp_ref[...], we0b_ref[...],
                                 preferred_element_type=jnp.float32)
                   ).astype(jnp.bfloat16)


def _wc(wp, we0b):
    return pl.pallas_call(
        _wc_body,
        out_shape=jax.ShapeDtypeStruct((D, D), jnp.bfloat16),
    )(wp, we0b)


CH = 640
NCH = NK // CH
QR = CH // 128
_NC = 2
_NS = 16
_NW = _NC * _NS


def _gather_body(sel_hbm, n2et_hbm, adja_hbm, adjb_hbm, fw_hbm, ew_hbm,
                 ga_hbm, gb_hbm, ge_hbm,
                 sel_v, fl_v, idx_v, an_v, bn_v,
                 arows_v, brows_v, erows_v, sem_idx, sem_el, sem_row,
                 sem_w):
    wid = lax.axis_index("s") * _NC + lax.axis_index("c")
    nt = (NCH - wid + _NW - 1) // _NW

    def fetch_idx(c):
        base = c * CH
        pltpu.sync_copy(sel_hbm.at[pl.ds(base, CH)], sel_v)
        for j in range(CH // 16):
            it = lax.iota(jnp.int32, 16) + (base + j * 16)
            fl_v[pl.ds(j * 16, 16)] = (
                sel_v[pl.ds(j * 16, 16)] * N
                + lax.shift_right_logical(it, 3))
        return [pltpu.async_copy(n2et_hbm.at[fl_v.at[pl.ds(q * 128, 128)]],
                                 idx_v.at[pl.ds(q * 128, 128)], sem_idx)
                for q in range(QR)]

    for cp in fetch_idx(wid):
        cp.wait()

    def drain_writes():
        pltpu.make_async_copy(arows_v, ga_hbm.at[pl.ds(0, CH)], sem_w).wait()
        pltpu.make_async_copy(brows_v, gb_hbm.at[pl.ds(0, CH)], sem_w).wait()
        pltpu.make_async_copy(erows_v, ge_hbm.at[pl.ds(0, CH)], sem_w).wait()

    def body(t, carry):
        c = wid + t * _NW
        base = c * CH

        @pl.when(t > 0)
        def _():
            drain_writes()

        ecps = [pltpu.async_copy(ew_hbm.at[idx_v.at[pl.ds(q * 128, 128)]],
                                 erows_v.at[pl.ds(q * 128, 128)], sem_row)
                for q in range(QR)]
        cps = [pltpu.async_copy(adja_hbm.at[idx_v.at[pl.ds(q * 128, 128)]],
                                an_v.at[pl.ds(q * 128, 128)], sem_el)
               for q in range(QR)]
        cps += [pltpu.async_copy(adjb_hbm.at[idx_v.at[pl.ds(q * 128, 128)]],
                                 bn_v.at[pl.ds(q * 128, 128)], sem_el)
                for q in range(QR)]
        for cp in cps:
            cp.wait()
        rcps = [pltpu.async_copy(fw_hbm.at[an_v.at[pl.ds(q * 128, 128)]],
                                 arows_v.at[pl.ds(q * 128, 128)], sem_row)
                for q in range(QR)]
        rcps += [pltpu.async_copy(fw_hbm.at[bn_v.at[pl.ds(q * 128, 128)]],
                                  brows_v.at[pl.ds(q * 128, 128)], sem_row)
                 for q in range(QR)]
        for cp in rcps + ecps:
            cp.wait()
        pltpu.async_copy(erows_v, ge_hbm.at[pl.ds(base, CH)], sem_w)
        pltpu.async_copy(arows_v, ga_hbm.at[pl.ds(base, CH)], sem_w)
        pltpu.async_copy(brows_v, gb_hbm.at[pl.ds(base, CH)], sem_w)

        @pl.when(t + 1 < nt)
        def _():
            for cp in fetch_idx(c + _NW):
                cp.wait()

        return carry

    lax.fori_loop(0, nt, body, 0)
    drain_writes()


def _gather_sc(sel1, n2et1, adja, adjb, fw, ew):
    mesh = plsc.VectorSubcoreMesh(core_axis_name="c", subcore_axis_name="s")
    return pl.kernel(
        _gather_body,
        mesh=mesh,
        compiler_params=pltpu.CompilerParams(use_tc_tiling_on_sc=False),
        out_type=(
            jax.ShapeDtypeStruct((NK, DW), jnp.int32),
            jax.ShapeDtypeStruct((NK, DW), jnp.int32),
            jax.ShapeDtypeStruct((NK, DW), jnp.int32),
        ),
        scratch_types=[
            pltpu.VMEM((CH,), jnp.int32),
            pltpu.VMEM((CH,), jnp.int32),
            pltpu.VMEM((CH,), jnp.int32),
            pltpu.VMEM((CH,), jnp.int32),
            pltpu.VMEM((CH,), jnp.int32),
            pltpu.VMEM((CH, DW), jnp.int32),
            pltpu.VMEM((CH, DW), jnp.int32),
            pltpu.VMEM((CH, DW), jnp.int32),
            pltpu.SemaphoreType.DMA,
            pltpu.SemaphoreType.DMA,
            pltpu.SemaphoreType.DMA,
            pltpu.SemaphoreType.DMA,
        ],
    )(sel1, n2et1, adja, adjb, fw, ew)


BN3 = 400
BR3 = BN3 * K // 2
NB3 = N // BN3


def _heads_body(ga_ref, gb_ref, ge_ref, f0_ref, we0a_ref, wc_ref,
                wn0a_ref, wn0b_ref, wn1a_ref, wn1b_ref, out_ref):
    xa = ga_ref[...]
    xb = gb_ref[...]
    xe = ge_ref[...]
    sl = _lo(xa) + _lo(xb)
    sh = _hi(xa) + _hi(xb)
    el = _lo(xe)
    eh = _hi(xe)
    slb = sl.astype(jnp.bfloat16)
    shb = sh.astype(jnp.bfloat16)
    elb = el.astype(jnp.bfloat16)
    ehb = eh.astype(jnp.bfloat16)
    we0a = we0a_ref[...]
    wc = wc_ref[...]

    def edge_head(cols):
        return jnp.maximum(
            jnp.dot(elb[:, cols], we0a[:DW],
                    preferred_element_type=jnp.float32)
            + jnp.dot(ehb[:, cols], we0a[DW:],
                      preferred_element_type=jnp.float32)
            + jnp.dot(slb[:, cols], wc[:DW],
                      preferred_element_type=jnp.float32)
            + jnp.dot(shb[:, cols], wc[DW:],
                      preferred_element_type=jnp.float32),
            0.0)

    eo_e = edge_head(slice(0, DW))
    eo_o = edge_head(slice(DW, D))

    def s4(x):
        x4 = x.reshape(BN3, 4 * D)
        return (x4[:, :D] + x4[:, D:2 * D] + x4[:, 2 * D:3 * D]
                + x4[:, 3 * D:])

    em1 = (s4(eo_e) + s4(eo_o)) * (1.0 / K)
    el4 = s4(el)
    eh4 = s4(eh)
    em0l = (el4[:, :DW] + el4[:, DW:]) * (1.0 / K)
    em0h = (eh4[:, :DW] + eh4[:, DW:]) * (1.0 / K)
    f0 = f0_ref[...]
    f1 = jnp.maximum(
        jnp.dot(f0, wn0a_ref[...], preferred_element_type=jnp.float32)
        + jnp.dot(em0l, wn0b_ref[:DW], preferred_element_type=jnp.float32)
        + jnp.dot(em0h, wn0b_ref[DW:], preferred_element_type=jnp.float32),
        0.0)
    f2 = jnp.maximum(
        jnp.dot(f1, wn1a_ref[...], preferred_element_type=jnp.float32)
        + jnp.dot(em1, wn1b_ref[...], preferred_element_type=jnp.float32),
        0.0)
    out_ref[...] = jnp.concatenate([f1, f2], axis=-1)[None]


def _heads(ga2, gb2, ge2, f0, we0a, wc, wn0a, wn0b, wn1a, wn1b):
    return pl.pallas_call(
        _heads_body,
        grid=(NB3,),
        in_specs=[
            pl.BlockSpec((BR3, D), lambda i: (i, 0)),
            pl.BlockSpec((BR3, D), lambda i: (i, 0)),
            pl.BlockSpec((BR3, D), lambda i: (i, 0)),
            pl.BlockSpec((BN3, D), lambda i: (i, 0)),
            pl.BlockSpec((D, D), lambda i: (0, 0)),
            pl.BlockSpec((D, D), lambda i: (0, 0)),
            pl.BlockSpec((D, D), lambda i: (0, 0)),
            pl.BlockSpec((D, D), lambda i: (0, 0)),
            pl.BlockSpec((D, D), lambda i: (0, 0)),
            pl.BlockSpec((D, D), lambda i: (0, 0)),
        ],
        out_specs=pl.BlockSpec((1, BN3, 2 * D), lambda i: (0, i, 0)),
        out_shape=jax.ShapeDtypeStruct((1, N, 2 * D), jnp.float32),
    )(ga2, gb2, ge2, f0, we0a, wc, wn0a, wn0b, wn1a, wn1b)


def kernel(feats, node2edge_idx, edge_emb, edge_node_adj, sel, W_prep,
           W_edge_prep, W_e0, W_n0, W_e1, W_n1):
    del W_e1
    f0, fw128 = _f0(feats, W_prep)
    ew128 = _e0(edge_emb.T, W_edge_prep)
    wc = _wc(W_prep, W_e0[D:])
    sel1 = sel.reshape(NK)
    n2et1 = node2edge_idx.T.reshape(N * S)
    adjt = edge_node_adj.T
    adja = adjt[0]
    adjb = adjt[1]
    fw = fw128.reshape(N, DW)
    ew = ew128.reshape(E, DW)
    gaw, gbw, gew = _gather_sc(sel1, n2et1, adja, adjb, fw, ew)
    ga2 = gaw.reshape(NK // 2, D)
    gb2 = gbw.reshape(NK // 2, D)
    ge2 = gew.reshape(NK // 2, D)
    return _heads(ga2, gb2, ge2, f0, W_e0[:D].astype(jnp.bfloat16), wc,
                  W_n0[:D], W_n0[D:], W_n1[:D], W_n1[D:])


# speedup vs baseline: 5.4134x; 1.0346x over previous
import functools

import jax
import jax.numpy as jnp
from jax import lax
from jax.experimental import pallas as pl
from jax.experimental.pallas import tpu as pltpu
from jax.experimental.pallas import tpu_sc as plsc

N = 50000
S = 16
E = N * S // 2
D = 128
ED = 16
K = 8
NK = N * K
DW = D // 2


def _pack64(f):
    lo = lax.bitcast_convert_type(f[:, :DW].astype(jnp.bfloat16), jnp.uint16)
    hi = lax.bitcast_convert_type(f[:, DW:].astype(jnp.bfloat16), jnp.uint16)
    return lo.astype(jnp.int32) | jnp.left_shift(hi.astype(jnp.int32), 16)


def _pack_pair_rows(f):
    f4 = f.reshape(f.shape[0] // 2, 2 * D)
    return jnp.concatenate([_pack64(f4[:, :D]), _pack64(f4[:, D:])],
                           axis=-1)


def _lo(x):
    return lax.bitcast_convert_type(jnp.left_shift(x, 16), jnp.float32)


def _hi(x):
    return lax.bitcast_convert_type(x & jnp.int32(-65536), jnp.float32)


BN1 = 2000
NB1 = N // BN1


def _f0_body(feats_ref, wp_ref, f0_ref, fw_ref):
    f = feats_ref[...]
    f0_ref[...] = jnp.dot(f, wp_ref[...], preferred_element_type=jnp.float32)
    fw_ref[...] = _pack_pair_rows(f)


def _f0(feats, wp):
    return pl.pallas_call(
        _f0_body,
        grid=(NB1,),
        in_specs=[
            pl.BlockSpec((BN1, D), lambda i: (i, 0)),
            pl.BlockSpec((D, D), lambda i: (0, 0)),
        ],
        out_specs=[
            pl.BlockSpec((BN1, D), lambda i: (i, 0)),
            pl.BlockSpec((BN1 // 2, D), lambda i: (i, 0)),
        ],
        out_shape=[
            jax.ShapeDtypeStruct((N, D), jnp.float32),
            jax.ShapeDtypeStruct((N // 2, D), jnp.int32),
        ],
    )(feats, wp)


BNE = 3200
NBE = E // BNE


def _e0_body(embt_ref, wep_ref, ew_ref):
    e0 = lax.dot_general(embt_ref[...], wep_ref[...],
                         (((0,), (0,)), ((), ())),
                         preferred_element_type=jnp.float32)
    ew_ref[...] = _pack_pair_rows(e0)


def _e0(embt, wep):
    return pl.pallas_call(
        _e0_body,
        grid=(NBE,),
        in_specs=[
            pl.BlockSpec((ED, BNE), lambda i: (0, i)),
            pl.BlockSpec((ED, D), lambda i: (0, 0)),
        ],
        out_specs=pl.BlockSpec((BNE // 2, D), lambda i: (i, 0)),
        out_shape=jax.ShapeDtypeStruct((E // 2, D), jnp.int32),
    )(embt, wep)


def _wc_body(wp_ref, we0b_ref, wc_ref):
    wc_ref[...] = (0.5 * jnp.dot(wp_ref[...], we0b_ref[...],
                                 preferred_element_type=jnp.float32)
                   ).astype(jnp.bfloat16)


def _wc(wp, we0b):
    return pl.pallas_call(
        _wc_body,
        out_shape=jax.ShapeDtypeStruct((D, D), jnp.bfloat16),
    )(wp, we0b)


CH = 640
NCH = NK // CH
QR = CH // 128
_SPLITS = (310, 315)
_NC = 2
_NS = 16
_NW = _NC * _NS


def _gather_body(c0, ncv, sel_hbm, n2et_hbm, adja_hbm, adjb_hbm, fw_hbm,
                 ew_hbm, ga_hbm, gb_hbm, ge_hbm,
                 sel_v, fl_v, idx_v, an_v, bn_v,
                 arows_v, brows_v, erows_v, sem_idx, sem_el, sem_row,
                 sem_w):
    wid = lax.axis_index("s") * _NC + lax.axis_index("c")
    nt = (ncv - wid + _NW - 1) // _NW

    def fetch_idx(c):
        base = (c0 + c) * CH
        pltpu.sync_copy(sel_hbm.at[pl.ds(base, CH)], sel_v)
        for j in range(CH // 16):
            it = lax.iota(jnp.int32, 16) + (base + j * 16)
            fl_v[pl.ds(j * 16, 16)] = (
                sel_v[pl.ds(j * 16, 16)] * N
                + lax.shift_right_logical(it, 3))
        return [pltpu.async_copy(n2et_hbm.at[fl_v.at[pl.ds(q * 128, 128)]],
                                 idx_v.at[pl.ds(q * 128, 128)], sem_idx)
                for q in range(QR)]

    for cp in fetch_idx(wid):
        cp.wait()

    def drain_writes():
        pltpu.make_async_copy(arows_v, ga_hbm.at[pl.ds(0, CH)], sem_w).wait()
        pltpu.make_async_copy(brows_v, gb_hbm.at[pl.ds(0, CH)], sem_w).wait()
        pltpu.make_async_copy(erows_v, ge_hbm.at[pl.ds(0, CH)], sem_w).wait()

    def body(t, carry):
        c = wid + t * _NW
        base = c * CH

        @pl.when(t > 0)
        def _():
            drain_writes()

        ecps = [pltpu.async_copy(ew_hbm.at[idx_v.at[pl.ds(q * 128, 128)]],
                                 erows_v.at[pl.ds(q * 128, 128)], sem_row)
                for q in range(QR)]
        cps = [pltpu.async_copy(adja_hbm.at[idx_v.at[pl.ds(q * 128, 128)]],
                                an_v.at[pl.ds(q * 128, 128)], sem_el)
               for q in range(QR)]
        cps += [pltpu.async_copy(adjb_hbm.at[idx_v.at[pl.ds(q * 128, 128)]],
                                 bn_v.at[pl.ds(q * 128, 128)], sem_el)
                for q in range(QR)]
        for cp in cps:
            cp.wait()
        rcps = [pltpu.async_copy(fw_hbm.at[an_v.at[pl.ds(q * 128, 128)]],
                                 arows_v.at[pl.ds(q * 128, 128)], sem_row)
                for q in range(QR)]
        rcps += [pltpu.async_copy(fw_hbm.at[bn_v.at[pl.ds(q * 128, 128)]],
                                  brows_v.at[pl.ds(q * 128, 128)], sem_row)
                 for q in range(QR)]
        for cp in rcps + ecps:
            cp.wait()
        pltpu.async_copy(erows_v, ge_hbm.at[pl.ds(base, CH)], sem_w)
        pltpu.async_copy(arows_v, ga_hbm.at[pl.ds(base, CH)], sem_w)
        pltpu.async_copy(brows_v, gb_hbm.at[pl.ds(base, CH)], sem_w)

        @pl.when(t + 1 < nt)
        def _():
            for cp in fetch_idx(c + _NW):
                cp.wait()

        return carry

    lax.fori_loop(0, nt, body, 0)
    drain_writes()


def _gather_sc(sel1, n2et1, adja, adjb, fw, ew, c0, nch):
    nk = nch * CH
    mesh = plsc.VectorSubcoreMesh(core_axis_name="c", subcore_axis_name="s")
    return pl.kernel(
        functools.partial(_gather_body, c0, nch),
        mesh=mesh,
        compiler_params=pltpu.CompilerParams(use_tc_tiling_on_sc=False),
        out_type=(
            jax.ShapeDtypeStruct((nk, DW), jnp.int32),
            jax.ShapeDtypeStruct((nk, DW), jnp.int32),
            jax.ShapeDtypeStruct((nk, DW), jnp.int32),
        ),
        scratch_types=[
            pltpu.VMEM((CH,), jnp.int32),
            pltpu.VMEM((CH,), jnp.int32),
            pltpu.VMEM((CH,), jnp.int32),
            pltpu.VMEM((CH,), jnp.int32),
            pltpu.VMEM((CH,), jnp.int32),
            pltpu.VMEM((CH, DW), jnp.int32),
            pltpu.VMEM((CH, DW), jnp.int32),
            pltpu.VMEM((CH, DW), jnp.int32),
            pltpu.SemaphoreType.DMA,
            pltpu.SemaphoreType.DMA,
            pltpu.SemaphoreType.DMA,
            pltpu.SemaphoreType.DMA,
        ],
    )(sel1, n2et1, adja, adjb, fw, ew)


BN3 = 400
BR3 = BN3 * K // 2
NB3 = N // BN3


def _heads_body(ga_ref, gb_ref, ge_ref, f0_ref, we0a_ref, wc_ref,
                wn0a_ref, wn0b_ref, wn1a_ref, wn1b_ref, out_ref):
    xa = ga_ref[...]
    xb = gb_ref[...]
    xe = ge_ref[...]
    sl = _lo(xa) + _lo(xb)
    sh = _hi(xa) + _hi(xb)
    el = _lo(xe)
    eh = _hi(xe)
    slb = sl.astype(jnp.bfloat16)
    shb = sh.astype(jnp.bfloat16)
    elb = el.astype(jnp.bfloat16)
    ehb = eh.astype(jnp.bfloat16)
    we0a = we0a_ref[...]
    wc = wc_ref[...]

    def edge_head(cols):
        return jnp.maximum(
            jnp.dot(elb[:, cols], we0a[:DW],
                    preferred_element_type=jnp.float32)
            + jnp.dot(ehb[:, cols], we0a[DW:],
                      preferred_element_type=jnp.float32)
            + jnp.dot(slb[:, cols], wc[:DW],
                      preferred_element_type=jnp.float32)
            + jnp.dot(shb[:, cols], wc[DW:],
                      preferred_element_type=jnp.float32),
            0.0)

    eo_e = edge_head(slice(0, DW))
    eo_o = edge_head(slice(DW, D))

    def s4(x):
        x4 = x.reshape(BN3, 4 * D)
        return (x4[:, :D] + x4[:, D:2 * D] + x4[:, 2 * D:3 * D]
                + x4[:, 3 * D:])

    em1 = (s4(eo_e) + s4(eo_o)) * (1.0 / K)
    el4 = s4(el)
    eh4 = s4(eh)
    em0l = (el4[:, :DW] + el4[:, DW:]) * (1.0 / K)
    em0h = (eh4[:, :DW] + eh4[:, DW:]) * (1.0 / K)
    f0 = f0_ref[...]
    f1 = jnp.maximum(
        jnp.dot(f0, wn0a_ref[...], preferred_element_type=jnp.float32)
        + jnp.dot(em0l, wn0b_ref[:DW], preferred_element_type=jnp.float32)
        + jnp.dot(em0h, wn0b_ref[DW:], preferred_element_type=jnp.float32),
        0.0)
    f2 = jnp.maximum(
        jnp.dot(f1, wn1a_ref[...], preferred_element_type=jnp.float32)
        + jnp.dot(em1, wn1b_ref[...], preferred_element_type=jnp.float32),
        0.0)
    out_ref[...] = jnp.concatenate([f1, f2], axis=-1)[None]


def _heads(ga2, gb2, ge2, f0, we0a, wc, wn0a, wn0b, wn1a, wn1b, b0, nb):
    return pl.pallas_call(
        _heads_body,
        grid=(nb,),
        in_specs=[
            pl.BlockSpec((BR3, D), lambda i: (i, 0)),
            pl.BlockSpec((BR3, D), lambda i: (i, 0)),
            pl.BlockSpec((BR3, D), lambda i: (i, 0)),
            pl.BlockSpec((BN3, D), lambda i: (i + b0, 0)),
            pl.BlockSpec((D, D), lambda i: (0, 0)),
            pl.BlockSpec((D, D), lambda i: (0, 0)),
            pl.BlockSpec((D, D), lambda i: (0, 0)),
            pl.BlockSpec((D, D), lambda i: (0, 0)),
            pl.BlockSpec((D, D), lambda i: (0, 0)),
            pl.BlockSpec((D, D), lambda i: (0, 0)),
        ],
        out_specs=pl.BlockSpec((1, BN3, 2 * D), lambda i: (0, i, 0)),
        out_shape=jax.ShapeDtypeStruct((1, nb * BN3, 2 * D), jnp.float32),
    )(ga2, gb2, ge2, f0, we0a, wc, wn0a, wn0b, wn1a, wn1b)


def kernel(feats, node2edge_idx, edge_emb, edge_node_adj, sel, W_prep,
           W_edge_prep, W_e0, W_n0, W_e1, W_n1):
    del W_e1
    f0, fw128 = _f0(feats, W_prep)
    ew128 = _e0(edge_emb.T, W_edge_prep)
    wc = _wc(W_prep, W_e0[D:])
    sel1 = sel.reshape(NK)
    n2et1 = node2edge_idx.T.reshape(N * S)
    adjt = edge_node_adj.T
    adja = adjt[0]
    adjb = adjt[1]
    fw = fw128.reshape(N, DW)
    ew = ew128.reshape(E, DW)
    we0a_bf = W_e0[:D].astype(jnp.bfloat16)
    wn0a, wn0b = W_n0[:D], W_n0[D:]
    wn1a, wn1b = W_n1[:D], W_n1[D:]
    parts = []
    c0 = 0
    for nch in _SPLITS:
        gaw, gbw, gew = _gather_sc(sel1, n2et1, adja, adjb, fw, ew, c0, nch)
        nk = nch * CH
        parts.append((_heads(gaw.reshape(nk // 2, D), gbw.reshape(nk // 2, D),
                             gew.reshape(nk // 2, D), f0, we0a_bf, wc,
                             wn0a, wn0b, wn1a, wn1b,
                             c0 * CH // (BN3 * K), nk // (BN3 * K))))
        c0 += nch
    return jnp.concatenate(parts, axis=1)


# speedup vs baseline: 5.7154x; 1.0558x over previous
import functools

import jax
import jax.numpy as jnp
from jax import lax
from jax.experimental import pallas as pl
from jax.experimental.pallas import tpu as pltpu
from jax.experimental.pallas import tpu_sc as plsc

N = 50000
S = 16
E = N * S // 2
D = 128
ED = 16
K = 8
NK = N * K
DW = D // 2


def _pack64(f):
    lo = lax.bitcast_convert_type(f[:, :DW].astype(jnp.bfloat16), jnp.uint16)
    hi = lax.bitcast_convert_type(f[:, DW:].astype(jnp.bfloat16), jnp.uint16)
    return lo.astype(jnp.int32) | jnp.left_shift(hi.astype(jnp.int32), 16)


def _pack_pair_rows(f):
    f4 = f.reshape(f.shape[0] // 2, 2 * D)
    return jnp.concatenate([_pack64(f4[:, :D]), _pack64(f4[:, D:])],
                           axis=-1)


def _lo(x):
    return lax.bitcast_convert_type(jnp.left_shift(x, 16), jnp.float32)


def _hi(x):
    return lax.bitcast_convert_type(x & jnp.int32(-65536), jnp.float32)


BN1 = 2000
NB1 = N // BN1


def _f0_body(feats_ref, wp_ref, f0_ref, fw_ref):
    f = feats_ref[...]
    f0_ref[...] = jnp.dot(f, wp_ref[...], preferred_element_type=jnp.float32)
    fw_ref[...] = _pack_pair_rows(f)


def _f0(feats, wp):
    return pl.pallas_call(
        _f0_body,
        grid=(NB1,),
        in_specs=[
            pl.BlockSpec((BN1, D), lambda i: (i, 0)),
            pl.BlockSpec((D, D), lambda i: (0, 0)),
        ],
        out_specs=[
            pl.BlockSpec((BN1, D), lambda i: (i, 0)),
            pl.BlockSpec((BN1 // 2, D), lambda i: (i, 0)),
        ],
        out_shape=[
            jax.ShapeDtypeStruct((N, D), jnp.float32),
            jax.ShapeDtypeStruct((N // 2, D), jnp.int32),
        ],
    )(feats, wp)


BNE = 16000
NBE = E // BNE


def _e0_body(embt_ref, wep_ref, ew_ref):
    e0 = lax.dot_general(embt_ref[...], wep_ref[...],
                         (((0,), (0,)), ((), ())),
                         preferred_element_type=jnp.float32)
    ew_ref[...] = _pack_pair_rows(e0)


def _e0(embt, wep):
    return pl.pallas_call(
        _e0_body,
        grid=(NBE,),
        in_specs=[
            pl.BlockSpec((ED, BNE), lambda i: (0, i)),
            pl.BlockSpec((ED, D), lambda i: (0, 0)),
        ],
        out_specs=pl.BlockSpec((BNE // 2, D), lambda i: (i, 0)),
        out_shape=jax.ShapeDtypeStruct((E // 2, D), jnp.int32),
    )(embt, wep)


def _wc_body(wp_ref, we0b_ref, wc_ref):
    wc_ref[...] = (0.5 * jnp.dot(wp_ref[...], we0b_ref[...],
                                 preferred_element_type=jnp.float32)
                   ).astype(jnp.bfloat16)


def _wc(wp, we0b):
    return pl.pallas_call(
        _wc_body,
        out_shape=jax.ShapeDtypeStruct((D, D), jnp.bfloat16),
    )(wp, we0b)


CH = 640
NCH = NK // CH
QR = CH // 128
_SPLITS = (30, 190, 190, 185, 30)
_NC = 2
_NS = 16
_NW = _NC * _NS


def _gather_body(c0, ncv, sel_hbm, n2et_hbm, adja_hbm, adjb_hbm, fw_hbm,
                 ew_hbm, ga_hbm, gb_hbm, ge_hbm,
                 sel_v, fl_v, idx_v, an_v, bn_v,
                 arows_v, brows_v, erows_v, sem_idx, sem_el, sem_row,
                 sem_w):
    wid = lax.axis_index("s") * _NC + lax.axis_index("c")
    nt = (ncv - wid + _NW - 1) // _NW

    def fetch_idx(c):
        base = (c0 + c) * CH
        pltpu.sync_copy(sel_hbm.at[pl.ds(base, CH)], sel_v)
        for j in range(CH // 16):
            it = lax.iota(jnp.int32, 16) + (base + j * 16)
            fl_v[pl.ds(j * 16, 16)] = (
                sel_v[pl.ds(j * 16, 16)] * N
                + lax.shift_right_logical(it, 3))
        return [pltpu.async_copy(n2et_hbm.at[fl_v.at[pl.ds(q * 128, 128)]],
                                 idx_v.at[pl.ds(q * 128, 128)], sem_idx)
                for q in range(QR)]

    for cp in fetch_idx(jnp.minimum(wid, ncv - 1)):
        cp.wait()

    def drain_writes():
        pltpu.make_async_copy(arows_v, ga_hbm.at[pl.ds(0, CH)], sem_w).wait()
        pltpu.make_async_copy(brows_v, gb_hbm.at[pl.ds(0, CH)], sem_w).wait()
        pltpu.make_async_copy(erows_v, ge_hbm.at[pl.ds(0, CH)], sem_w).wait()

    def body(t, carry):
        c = wid + t * _NW
        base = c * CH

        @pl.when(t > 0)
        def _():
            drain_writes()

        ecps = [pltpu.async_copy(ew_hbm.at[idx_v.at[pl.ds(q * 128, 128)]],
                                 erows_v.at[pl.ds(q * 128, 128)], sem_row)
                for q in range(QR)]
        cps = [pltpu.async_copy(adja_hbm.at[idx_v.at[pl.ds(q * 128, 128)]],
                                an_v.at[pl.ds(q * 128, 128)], sem_el)
               for q in range(QR)]
        cps += [pltpu.async_copy(adjb_hbm.at[idx_v.at[pl.ds(q * 128, 128)]],
                                 bn_v.at[pl.ds(q * 128, 128)], sem_el)
                for q in range(QR)]
        for cp in cps:
            cp.wait()
        rcps = [pltpu.async_copy(fw_hbm.at[an_v.at[pl.ds(q * 128, 128)]],
                                 arows_v.at[pl.ds(q * 128, 128)], sem_row)
                for q in range(QR)]
        rcps += [pltpu.async_copy(fw_hbm.at[bn_v.at[pl.ds(q * 128, 128)]],
                                  brows_v.at[pl.ds(q * 128, 128)], sem_row)
                 for q in range(QR)]
        for cp in rcps + ecps:
            cp.wait()
        pltpu.async_copy(erows_v, ge_hbm.at[pl.ds(base, CH)], sem_w)
        pltpu.async_copy(arows_v, ga_hbm.at[pl.ds(base, CH)], sem_w)
        pltpu.async_copy(brows_v, gb_hbm.at[pl.ds(base, CH)], sem_w)

        @pl.when(t + 1 < nt)
        def _():
            for cp in fetch_idx(c + _NW):
                cp.wait()

        return carry

    lax.fori_loop(0, nt, body, 0)

    @pl.when(nt > 0)
    def _():
        drain_writes()


def _gather_sc(sel1, n2et1, adja, adjb, fw, ew, c0, nch):
    nk = nch * CH
    mesh = plsc.VectorSubcoreMesh(core_axis_name="c", subcore_axis_name="s")
    return pl.kernel(
        functools.partial(_gather_body, c0, nch),
        mesh=mesh,
        compiler_params=pltpu.CompilerParams(use_tc_tiling_on_sc=False),
        out_type=(
            jax.ShapeDtypeStruct((nk, DW), jnp.int32),
            jax.ShapeDtypeStruct((nk, DW), jnp.int32),
            jax.ShapeDtypeStruct((nk, DW), jnp.int32),
        ),
        scratch_types=[
            pltpu.VMEM((CH,), jnp.int32),
            pltpu.VMEM((CH,), jnp.int32),
            pltpu.VMEM((CH,), jnp.int32),
            pltpu.VMEM((CH,), jnp.int32),
            pltpu.VMEM((CH,), jnp.int32),
            pltpu.VMEM((CH, DW), jnp.int32),
            pltpu.VMEM((CH, DW), jnp.int32),
            pltpu.VMEM((CH, DW), jnp.int32),
            pltpu.SemaphoreType.DMA,
            pltpu.SemaphoreType.DMA,
            pltpu.SemaphoreType.DMA,
            pltpu.SemaphoreType.DMA,
        ],
    )(sel1, n2et1, adja, adjb, fw, ew)


BN3 = 400
BR3 = BN3 * K // 2
NB3 = N // BN3


def _heads_body(ga_ref, gb_ref, ge_ref, f0_ref, we0a_ref, wc_ref,
                wn0a_ref, wn0b_ref, wn1a_ref, wn1b_ref, out_ref):
    xa = ga_ref[...]
    xb = gb_ref[...]
    xe = ge_ref[...]
    sl = _lo(xa) + _lo(xb)
    sh = _hi(xa) + _hi(xb)
    el = _lo(xe)
    eh = _hi(xe)
    slb = sl.astype(jnp.bfloat16)
    shb = sh.astype(jnp.bfloat16)
    elb = el.astype(jnp.bfloat16)
    ehb = eh.astype(jnp.bfloat16)
    we0a = we0a_ref[...]
    wc = wc_ref[...]

    def edge_head(cols):
        return jnp.maximum(
            jnp.dot(elb[:, cols], we0a[:DW],
                    preferred_element_type=jnp.float32)
            + jnp.dot(ehb[:, cols], we0a[DW:],
                      preferred_element_type=jnp.float32)
            + jnp.dot(slb[:, cols], wc[:DW],
                      preferred_element_type=jnp.float32)
            + jnp.dot(shb[:, cols], wc[DW:],
                      preferred_element_type=jnp.float32),
            0.0)

    eo_e = edge_head(slice(0, DW))
    eo_o = edge_head(slice(DW, D))

    def s4(x):
        x4 = x.reshape(BN3, 4 * D)
        return (x4[:, :D] + x4[:, D:2 * D] + x4[:, 2 * D:3 * D]
                + x4[:, 3 * D:])

    em1 = (s4(eo_e) + s4(eo_o)) * (1.0 / K)
    el4 = s4(el)
    eh4 = s4(eh)
    em0l = (el4[:, :DW] + el4[:, DW:]) * (1.0 / K)
    em0h = (eh4[:, :DW] + eh4[:, DW:]) * (1.0 / K)
    f0 = f0_ref[...]
    f1 = jnp.maximum(
        jnp.dot(f0, wn0a_ref[...], preferred_element_type=jnp.float32)
        + jnp.dot(em0l, wn0b_ref[:DW], preferred_element_type=jnp.float32)
        + jnp.dot(em0h, wn0b_ref[DW:], preferred_element_type=jnp.float32),
        0.0)
    f2 = jnp.maximum(
        jnp.dot(f1, wn1a_ref[...], preferred_element_type=jnp.float32)
        + jnp.dot(em1, wn1b_ref[...], preferred_element_type=jnp.float32),
        0.0)
    out_ref[...] = jnp.concatenate([f1, f2], axis=-1)[None]


def _heads(ga2, gb2, ge2, f0, we0a, wc, wn0a, wn0b, wn1a, wn1b, b0, nb):
    return pl.pallas_call(
        _heads_body,
        grid=(nb,),
        in_specs=[
            pl.BlockSpec((BR3, D), lambda i: (i, 0)),
            pl.BlockSpec((BR3, D), lambda i: (i, 0)),
            pl.BlockSpec((BR3, D), lambda i: (i, 0)),
            pl.BlockSpec((BN3, D), lambda i: (i + b0, 0)),
            pl.BlockSpec((D, D), lambda i: (0, 0)),
            pl.BlockSpec((D, D), lambda i: (0, 0)),
            pl.BlockSpec((D, D), lambda i: (0, 0)),
            pl.BlockSpec((D, D), lambda i: (0, 0)),
            pl.BlockSpec((D, D), lambda i: (0, 0)),
            pl.BlockSpec((D, D), lambda i: (0, 0)),
        ],
        out_specs=pl.BlockSpec((1, BN3, 2 * D), lambda i: (0, i, 0)),
        out_shape=jax.ShapeDtypeStruct((1, nb * BN3, 2 * D), jnp.float32),
    )(ga2, gb2, ge2, f0, we0a, wc, wn0a, wn0b, wn1a, wn1b)


def kernel(feats, node2edge_idx, edge_emb, edge_node_adj, sel, W_prep,
           W_edge_prep, W_e0, W_n0, W_e1, W_n1):
    del W_e1
    f0, fw128 = _f0(feats, W_prep)
    ew128 = _e0(edge_emb.T, W_edge_prep)
    wc = _wc(W_prep, W_e0[D:])
    sel1 = sel.reshape(NK)
    n2et1 = node2edge_idx.T.reshape(N * S)
    adjt = edge_node_adj.T
    adja = adjt[0]
    adjb = adjt[1]
    fw = fw128.reshape(N, DW)
    ew = ew128.reshape(E, DW)
    we0a_bf = W_e0[:D].astype(jnp.bfloat16)
    wn0a, wn0b = W_n0[:D], W_n0[D:]
    wn1a, wn1b = W_n1[:D], W_n1[D:]
    parts = []
    c0 = 0
    for nch in _SPLITS:
        gaw, gbw, gew = _gather_sc(sel1, n2et1, adja, adjb, fw, ew, c0, nch)
        nk = nch * CH
        parts.append((_heads(gaw.reshape(nk // 2, D), gbw.reshape(nk // 2, D),
                             gew.reshape(nk // 2, D), f0, we0a_bf, wc,
                             wn0a, wn0b, wn1a, wn1b,
                             c0 * CH // (BN3 * K), nk // (BN3 * K))))
        c0 += nch
    return jnp.concatenate(parts, axis=1)


# speedup vs baseline: 5.7345x; 1.0033x over previous
import functools

import jax
import jax.numpy as jnp
from jax import lax
from jax.experimental import pallas as pl
from jax.experimental.pallas import tpu as pltpu
from jax.experimental.pallas import tpu_sc as plsc

N = 50000
S = 16
E = N * S // 2
D = 128
ED = 16
K = 8
NK = N * K
DW = D // 2


def _pack64(f):
    lo = lax.bitcast_convert_type(f[:, :DW].astype(jnp.bfloat16), jnp.uint16)
    hi = lax.bitcast_convert_type(f[:, DW:].astype(jnp.bfloat16), jnp.uint16)
    return lo.astype(jnp.int32) | jnp.left_shift(hi.astype(jnp.int32), 16)


def _pack_pair_rows(f):
    f4 = f.reshape(f.shape[0] // 2, 2 * D)
    return jnp.concatenate([_pack64(f4[:, :D]), _pack64(f4[:, D:])],
                           axis=-1)


def _lo(x):
    return lax.bitcast_convert_type(jnp.left_shift(x, 16), jnp.float32)


def _hi(x):
    return lax.bitcast_convert_type(x & jnp.int32(-65536), jnp.float32)


BN1 = 2000
NB1 = N // BN1


def _f0_body(feats_ref, wp_ref, f0_ref, fw_ref):
    f = feats_ref[...]
    f0_ref[...] = jnp.dot(f, wp_ref[...], preferred_element_type=jnp.float32)
    fw_ref[...] = _pack_pair_rows(f)


def _f0(feats, wp):
    return pl.pallas_call(
        _f0_body,
        grid=(NB1,),
        in_specs=[
            pl.BlockSpec((BN1, D), lambda i: (i, 0)),
            pl.BlockSpec((D, D), lambda i: (0, 0)),
        ],
        out_specs=[
            pl.BlockSpec((BN1, D), lambda i: (i, 0)),
            pl.BlockSpec((BN1 // 2, D), lambda i: (i, 0)),
        ],
        out_shape=[
            jax.ShapeDtypeStruct((N, D), jnp.float32),
            jax.ShapeDtypeStruct((N // 2, D), jnp.int32),
        ],
    )(feats, wp)


BNE = 16000
NBE = E // BNE


def _e0_body(embt_ref, wep_ref, ew_ref):
    e0 = lax.dot_general(embt_ref[...], wep_ref[...],
                         (((0,), (0,)), ((), ())),
                         preferred_element_type=jnp.float32)
    ew_ref[...] = _pack_pair_rows(e0)


def _e0(embt, wep):
    return pl.pallas_call(
        _e0_body,
        grid=(NBE,),
        in_specs=[
            pl.BlockSpec((ED, BNE), lambda i: (0, i)),
            pl.BlockSpec((ED, D), lambda i: (0, 0)),
        ],
        out_specs=pl.BlockSpec((BNE // 2, D), lambda i: (i, 0)),
        out_shape=jax.ShapeDtypeStruct((E // 2, D), jnp.int32),
    )(embt, wep)


def _wc_body(wp_ref, we0b_ref, warm_ref, wc_ref):
    del warm_ref
    wc_ref[...] = (0.5 * jnp.dot(wp_ref[...], we0b_ref[...],
                                 preferred_element_type=jnp.float32)
                   ).astype(jnp.bfloat16)


def _wc(wp, we0b, warm):
    return pl.pallas_call(
        _wc_body,
        out_shape=jax.ShapeDtypeStruct((D, D), jnp.bfloat16),
    )(wp, we0b, warm)


CH = 640
NCH = NK // CH
QR = CH // 128
_SPLITS = (30, 190, 190, 185, 30)
_NC = 2
_NS = 16
_NW = _NC * _NS


def _gather_body(c0, ncv, sel_hbm, n2et_hbm, adja_hbm, adjb_hbm, fw_hbm,
                 ew_hbm, ga_hbm, gb_hbm, ge_hbm,
                 sel_v, fl_v, idx_v, an_v, bn_v,
                 arows_v, brows_v, erows_v, sem_idx, sem_el, sem_row,
                 sem_w):
    wid = lax.axis_index("s") * _NC + lax.axis_index("c")
    nt = (ncv - wid + _NW - 1) // _NW

    def fetch_idx(c):
        base = (c0 + c) * CH
        pltpu.sync_copy(sel_hbm.at[pl.ds(base, CH)], sel_v)
        for j in range(CH // 16):
            it = lax.iota(jnp.int32, 16) + (base + j * 16)
            fl_v[pl.ds(j * 16, 16)] = (
                sel_v[pl.ds(j * 16, 16)] * N
                + lax.shift_right_logical(it, 3))
        return [pltpu.async_copy(n2et_hbm.at[fl_v.at[pl.ds(q * 128, 128)]],
                                 idx_v.at[pl.ds(q * 128, 128)], sem_idx)
                for q in range(QR)]

    for cp in fetch_idx(jnp.minimum(wid, ncv - 1)):
        cp.wait()

    def drain_writes():
        pltpu.make_async_copy(arows_v, ga_hbm.at[pl.ds(0, CH)], sem_w).wait()
        pltpu.make_async_copy(brows_v, gb_hbm.at[pl.ds(0, CH)], sem_w).wait()
        pltpu.make_async_copy(erows_v, ge_hbm.at[pl.ds(0, CH)], sem_w).wait()

    def body(t, carry):
        c = wid + t * _NW
        base = c * CH

        @pl.when(t > 0)
        def _():
            drain_writes()

        ecps = [pltpu.async_copy(ew_hbm.at[idx_v.at[pl.ds(q * 128, 128)]],
                                 erows_v.at[pl.ds(q * 128, 128)], sem_row)
                for q in range(QR)]
        cps = [pltpu.async_copy(adja_hbm.at[idx_v.at[pl.ds(q * 128, 128)]],
                                an_v.at[pl.ds(q * 128, 128)], sem_el)
               for q in range(QR)]
        cps += [pltpu.async_copy(adjb_hbm.at[idx_v.at[pl.ds(q * 128, 128)]],
                                 bn_v.at[pl.ds(q * 128, 128)], sem_el)
                for q in range(QR)]
        for cp in cps:
            cp.wait()
        rcps = [pltpu.async_copy(fw_hbm.at[an_v.at[pl.ds(q * 128, 128)]],
                                 arows_v.at[pl.ds(q * 128, 128)], sem_row)
                for q in range(QR)]
        rcps += [pltpu.async_copy(fw_hbm.at[bn_v.at[pl.ds(q * 128, 128)]],
                                  brows_v.at[pl.ds(q * 128, 128)], sem_row)
                 for q in range(QR)]
        for cp in rcps + ecps:
            cp.wait()
        pltpu.async_copy(erows_v, ge_hbm.at[pl.ds(base, CH)], sem_w)
        pltpu.async_copy(arows_v, ga_hbm.at[pl.ds(base, CH)], sem_w)
        pltpu.async_copy(brows_v, gb_hbm.at[pl.ds(base, CH)], sem_w)

        @pl.when(t + 1 < nt)
        def _():
            for cp in fetch_idx(c + _NW):
                cp.wait()

        return carry

    lax.fori_loop(0, nt, body, 0)

    @pl.when(nt > 0)
    def _():
        drain_writes()


def _warm_body(sel_hbm, out_hbm, buf_v, sem):
    wid = lax.axis_index("s") * _NC + lax.axis_index("c")

    @pl.when(wid == 0)
    def _():
        pltpu.async_copy(sel_hbm.at[pl.ds(0, 8)], buf_v, sem).wait()
        pltpu.async_copy(buf_v, out_hbm, sem).wait()


def _warm_sc(sel1):
    mesh = plsc.VectorSubcoreMesh(core_axis_name="c", subcore_axis_name="s")
    return pl.kernel(
        _warm_body,
        mesh=mesh,
        compiler_params=pltpu.CompilerParams(use_tc_tiling_on_sc=False),
        out_type=jax.ShapeDtypeStruct((8,), jnp.int32),
        scratch_types=[
            pltpu.VMEM((8,), jnp.int32),
            pltpu.SemaphoreType.DMA,
        ],
    )(sel1)


def _gather_sc(sel1, n2et1, adja, adjb, fw, ew, c0, nch):
    nk = nch * CH
    mesh = plsc.VectorSubcoreMesh(core_axis_name="c", subcore_axis_name="s")
    return pl.kernel(
        functools.partial(_gather_body, c0, nch),
        mesh=mesh,
        compiler_params=pltpu.CompilerParams(use_tc_tiling_on_sc=False),
        out_type=(
            jax.ShapeDtypeStruct((nk, DW), jnp.int32),
            jax.ShapeDtypeStruct((nk, DW), jnp.int32),
            jax.ShapeDtypeStruct((nk, DW), jnp.int32),
        ),
        scratch_types=[
            pltpu.VMEM((CH,), jnp.int32),
            pltpu.VMEM((CH,), jnp.int32),
            pltpu.VMEM((CH,), jnp.int32),
            pltpu.VMEM((CH,), jnp.int32),
            pltpu.VMEM((CH,), jnp.int32),
            pltpu.VMEM((CH, DW), jnp.int32),
            pltpu.VMEM((CH, DW), jnp.int32),
            pltpu.VMEM((CH, DW), jnp.int32),
            pltpu.SemaphoreType.DMA,
            pltpu.SemaphoreType.DMA,
            pltpu.SemaphoreType.DMA,
            pltpu.SemaphoreType.DMA,
        ],
    )(sel1, n2et1, adja, adjb, fw, ew)


BN3 = 400
BR3 = BN3 * K // 2
NB3 = N // BN3


def _heads_body(ga_ref, gb_ref, ge_ref, f0_ref, we0a_ref, wc_ref,
                wn0a_ref, wn0b_ref, wn1a_ref, wn1b_ref, out_ref):
    xa = ga_ref[...]
    xb = gb_ref[...]
    xe = ge_ref[...]
    sl = _lo(xa) + _lo(xb)
    sh = _hi(xa) + _hi(xb)
    el = _lo(xe)
    eh = _hi(xe)
    slb = sl.astype(jnp.bfloat16)
    shb = sh.astype(jnp.bfloat16)
    elb = el.astype(jnp.bfloat16)
    ehb = eh.astype(jnp.bfloat16)
    we0a = we0a_ref[...]
    wc = wc_ref[...]

    def edge_head(cols):
        return jnp.maximum(
            jnp.dot(elb[:, cols], we0a[:DW],
                    preferred_element_type=jnp.float32)
            + jnp.dot(ehb[:, cols], we0a[DW:],
                      preferred_element_type=jnp.float32)
            + jnp.dot(slb[:, cols], wc[:DW],
                      preferred_element_type=jnp.float32)
            + jnp.dot(shb[:, cols], wc[DW:],
                      preferred_element_type=jnp.float32),
            0.0)

    eo_e = edge_head(slice(0, DW))
    eo_o = edge_head(slice(DW, D))

    def s4(x):
        x4 = x.reshape(BN3, 4 * D)
        return (x4[:, :D] + x4[:, D:2 * D] + x4[:, 2 * D:3 * D]
                + x4[:, 3 * D:])

    em1 = (s4(eo_e) + s4(eo_o)) * (1.0 / K)
    el4 = s4(el)
    eh4 = s4(eh)
    em0l = (el4[:, :DW] + el4[:, DW:]) * (1.0 / K)
    em0h = (eh4[:, :DW] + eh4[:, DW:]) * (1.0 / K)
    f0 = f0_ref[...]
    f1 = jnp.maximum(
        jnp.dot(f0, wn0a_ref[...], preferred_element_type=jnp.float32)
        + jnp.dot(em0l, wn0b_ref[:DW], preferred_element_type=jnp.float32)
        + jnp.dot(em0h, wn0b_ref[DW:], preferred_element_type=jnp.float32),
        0.0)
    f2 = jnp.maximum(
        jnp.dot(f1, wn1a_ref[...], preferred_element_type=jnp.float32)
        + jnp.dot(em1, wn1b_ref[...], preferred_element_type=jnp.float32),
        0.0)
    out_ref[...] = jnp.concatenate([f1, f2], axis=-1)[None]


def _heads(ga2, gb2, ge2, f0, we0a, wc, wn0a, wn0b, wn1a, wn1b, b0, nb):
    return pl.pallas_call(
        _heads_body,
        grid=(nb,),
        in_specs=[
            pl.BlockSpec((BR3, D), lambda i: (i, 0)),
            pl.BlockSpec((BR3, D), lambda i: (i, 0)),
            pl.BlockSpec((BR3, D), lambda i: (i, 0)),
            pl.BlockSpec((BN3, D), lambda i: (i + b0, 0)),
            pl.BlockSpec((D, D), lambda i: (0, 0)),
            pl.BlockSpec((D, D), lambda i: (0, 0)),
            pl.BlockSpec((D, D), lambda i: (0, 0)),
            pl.BlockSpec((D, D), lambda i: (0, 0)),
            pl.BlockSpec((D, D), lambda i: (0, 0)),
            pl.BlockSpec((D, D), lambda i: (0, 0)),
        ],
        out_specs=pl.BlockSpec((1, BN3, 2 * D), lambda i: (0, i, 0)),
        out_shape=jax.ShapeDtypeStruct((1, nb * BN3, 2 * D), jnp.float32),
    )(ga2, gb2, ge2, f0, we0a, wc, wn0a, wn0b, wn1a, wn1b)


def kernel(feats, node2edge_idx, edge_emb, edge_node_adj, sel, W_prep,
           W_edge_prep, W_e0, W_n0, W_e1, W_n1):
    del W_e1
    f0, fw128 = _f0(feats, W_prep)
    ew128 = _e0(edge_emb.T, W_edge_prep)
    sel1 = sel.reshape(NK)
    wc = _wc(W_prep, W_e0[D:], _warm_sc(sel1))
    n2et1 = node2edge_idx.T.reshape(N * S)
    adjt = edge_node_adj.T
    adja = adjt[0]
    adjb = adjt[1]
    fw = fw128.reshape(N, DW)
    ew = ew128.reshape(E, DW)
    we0a_bf = W_e0[:D].astype(jnp.bfloat16)
    wn0a, wn0b = W_n0[:D], W_n0[D:]
    wn1a, wn1b = W_n1[:D], W_n1[D:]
    parts = []
    c0 = 0
    for nch in _SPLITS:
        gaw, gbw, gew = _gather_sc(sel1, n2et1, adja, adjb, fw, ew, c0, nch)
        nk = nch * CH
        parts.append((_heads(gaw.reshape(nk // 2, D), gbw.reshape(nk // 2, D),
                             gew.reshape(nk // 2, D), f0, we0a_bf, wc,
                             wn0a, wn0b, wn1a, wn1b,
                             c0 * CH // (BN3 * K), nk // (BN3 * K))))
        c0 += nch
    return jnp.concatenate(parts, axis=1)
